# Initial kernel scaffold; baseline (speedup 1.0000x reference)
import jax
import jax.numpy as jnp
from jax.experimental import pallas as pl


def kernel(person_ids, hobby_ids, edge_index, edge_weight, person_emb, hobby_emb):
    raise NotImplementedError("write your pallas kernel here")


# speedup vs baseline: 4.1877x; 4.1877x over previous
import functools

import jax
import jax.numpy as jnp
from jax import lax
from jax.experimental import pallas as pl
from jax.experimental.pallas import tpu as pltpu
from jax.experimental.pallas import tpu_sc as plsc

NUM_P = 60000
NUM_H = 40000
N = 100000
D = 32
E = 1600000
B = 4096
C = 128
NCHUNKS = E // C
NC = 2
NS = 16
HALF = N // NC
ZROWS = 400
NBLOCKS = HALF // ZROWS
PP = B // (NC * NS)

_mesh = plsc.VectorSubcoreMesh(core_axis_name="c", subcore_axis_name="s")

_GDN = lax.GatherDimensionNumbers(
    offset_dims=(), collapsed_slice_dims=(0,), start_index_map=(0,))


def _lane_bcast(vec, t):
    idx = jnp.full((16, 1), t, jnp.int32)
    return lax.gather(vec, idx, _GDN, slice_sizes=(1,),
                      mode=lax.GatherScatterMode.PROMISE_IN_BOUNDS)


def _propagate(dst, src, w, cur):
    @functools.partial(
        pl.kernel,
        out_type=jax.ShapeDtypeStruct((N, D), jnp.float32),
        mesh=_mesh,
        scratch_types=[
            pltpu.VMEM_SHARED((HALF, D), jnp.float32),
            pltpu.VMEM((ZROWS, D), jnp.float32),
            pltpu.VMEM((C,), jnp.int32),
            pltpu.VMEM((C,), jnp.int32),
            pltpu.VMEM((C,), jnp.float32),
            pltpu.VMEM((C,), jnp.int32),
            pltpu.VMEM((C, D), jnp.float32),
            pltpu.SemaphoreType.DMA,
        ],
        compiler_params=pltpu.CompilerParams(use_tc_tiling_on_sc=False, needs_layout_passes=False),
    )
    def k(dst_hbm, src_hbm, w_hbm, cur_hbm, out_hbm,
          accum, zbuf, dstb, srcb, wb, ldstb, rows, sem):
        cid = lax.axis_index("c")
        sid = lax.axis_index("s")
        zero16 = jnp.zeros((16,), jnp.float32)

        def zb(i, carry):
            zbuf[i, pl.ds(0, 16)] = zero16
            zbuf[i, pl.ds(16, 16)] = zero16
            return carry
        lax.fori_loop(0, ZROWS, zb, 0)

        nblocks_mine = (NBLOCKS - sid + NS - 1) // NS

        def zero_block(i, carry):
            pltpu.sync_copy(zbuf, accum.at[pl.ds((sid + i * NS) * ZROWS, ZROWS)])
            return carry
        lax.fori_loop(0, nblocks_mine, zero_block, 0)
        plsc.subcore_barrier()

        lo = cid * HALF

        def chunk_body(i, carry):
            ebase = (sid + i * NS) * C
            pltpu.sync_copy(dst_hbm.at[pl.ds(ebase, C)], dstb)
            pltpu.sync_copy(src_hbm.at[pl.ds(ebase, C)], srcb)
            pltpu.sync_copy(w_hbm.at[pl.ds(ebase, C)], wb)
            pltpu.async_copy(cur_hbm.at[srcb], rows, sem).wait()
            for v in range(C // 16):
                dv = dstb[pl.ds(v * 16, 16)]
                wv = wb[pl.ds(v * 16, 16)]
                m = (dv >= lo) & (dv < lo + HALF)
                ldstb[pl.ds(v * 16, 16)] = jnp.where(m, dv - lo, 0)
                wg = jnp.where(m, wv, jnp.zeros((16,), jnp.float32))
                for t in range(16):
                    j = v * 16 + t
                    ws = _lane_bcast(wg, t)
                    rows[j, pl.ds(0, 16)] = rows[j, pl.ds(0, 16)] * ws
                    rows[j, pl.ds(16, 16)] = rows[j, pl.ds(16, 16)] * ws
            pltpu.sync_copy(rows, accum.at[ldstb], add=True)
            return carry

        nchunks_mine = (NCHUNKS - sid + NS - 1) // NS
        lax.fori_loop(0, nchunks_mine, chunk_body, 0)

        plsc.subcore_barrier()

        def write_block(i, carry):
            r0 = (sid + i * NS) * ZROWS
            pltpu.sync_copy(accum.at[pl.ds(r0, ZROWS)],
                            out_hbm.at[pl.ds(cid * HALF + r0, ZROWS)])
            return carry
        lax.fori_loop(0, nblocks_mine, write_block, 0)

    return k(dst, src, w, cur)


def _score(pid, hid, t0, t1, t2, t3):
    @functools.partial(
        pl.kernel,
        out_type=jax.ShapeDtypeStruct((B,), jnp.float32),
        mesh=_mesh,
        scratch_types=[
            pltpu.VMEM((PP,), jnp.int32),
            pltpu.VMEM((PP,), jnp.int32),
            pltpu.VMEM((PP, D), jnp.float32),
            pltpu.VMEM((PP, D), jnp.float32),
            pltpu.VMEM((PP, D), jnp.float32),
            pltpu.VMEM((PP,), jnp.float32),
            pltpu.SemaphoreType.DMA,
        ],
        compiler_params=pltpu.CompilerParams(use_tc_tiling_on_sc=False, needs_layout_passes=False),
    )
    def k(pid_hbm, hid_hbm, t0_hbm, t1_hbm, t2_hbm, t3_hbm, out_hbm,
          pidb, hidb, pacc, hacc, tmp, sb, sem):
        cid = lax.axis_index("c")
        sid = lax.axis_index("s")
        base = (cid * NS + sid) * PP
        pltpu.sync_copy(pid_hbm.at[pl.ds(base, PP)], pidb)
        pltpu.sync_copy(hid_hbm.at[pl.ds(base, PP)], hidb)

        def off(i, carry):
            hidb[pl.ds(i * 16, 16)] = hidb[pl.ds(i * 16, 16)] + NUM_P
            return carry
        lax.fori_loop(0, PP // 16, off, 0)

        def accumulate(idxb, acc):
            pltpu.async_copy(t0_hbm.at[idxb], acc, sem).wait()
            for t_hbm in (t1_hbm, t2_hbm, t3_hbm):
                pltpu.async_copy(t_hbm.at[idxb], tmp, sem).wait()

                def addv(r, carry):
                    acc[r, pl.ds(0, 16)] = acc[r, pl.ds(0, 16)] + tmp[r, pl.ds(0, 16)]
                    acc[r, pl.ds(16, 16)] = acc[r, pl.ds(16, 16)] + tmp[r, pl.ds(16, 16)]
                    return carry
                lax.fori_loop(0, PP, addv, 0)

        accumulate(pidb, pacc)
        accumulate(hidb, hacc)

        iota = lax.iota(jnp.int32, 16)

        def dotg(g, carry):
            rowidx = g * 16 + iota
            acc = jnp.zeros((16,), jnp.float32)
            for d in range(D):
                col = jnp.full((16,), d, jnp.int32)
                pc = plsc.load_gather(pacc, [rowidx, col])
                hc = plsc.load_gather(hacc, [rowidx, col])
                acc = acc + pc * hc
            sb[pl.ds(g * 16, 16)] = acc * jnp.float32(1.0 / 16.0)
            return carry
        lax.fori_loop(0, PP // 16, dotg, 0)
        pltpu.sync_copy(sb, out_hbm.at[pl.ds(base, PP)])

    return k(pid, hid, t0, t1, t2, t3)


def kernel(person_ids, hobby_ids, edge_index, edge_weight, person_emb, hobby_emb):
    dst = edge_index[0].astype(jnp.int32)
    src = edge_index[1].astype(jnp.int32)
    w = edge_weight.astype(jnp.float32)
    t0 = jnp.concatenate([person_emb, hobby_emb], axis=0)
    t1 = _propagate(dst, src, w, t0)
    t2 = _propagate(dst, src, w, t1)
    t3 = _propagate(dst, src, w, t2)
    return _score(person_ids.astype(jnp.int32), hobby_ids.astype(jnp.int32),
                  t0, t1, t2, t3)


# speedup vs baseline: 8.3336x; 1.9900x over previous
import functools

import jax
import jax.numpy as jnp
from jax import lax
from jax.experimental import pallas as pl
from jax.experimental.pallas import tpu as pltpu
from jax.experimental.pallas import tpu_sc as plsc

NUM_P = 60000
NUM_H = 40000
N = 100000
D = 32
E = 1600000
B = 4096
C = 128
NCHUNKS = E // C
NC = 2
NS = 16
HALF = N // NC
ZROWS = 400
NBLOCKS = HALF // ZROWS
PP = B // (NC * NS)

_mesh = plsc.VectorSubcoreMesh(core_axis_name="c", subcore_axis_name="s")

_GDN = lax.GatherDimensionNumbers(
    offset_dims=(), collapsed_slice_dims=(0,), start_index_map=(0,))


def _lane_bcast(vec, t):
    idx = jnp.full((16, 1), t, jnp.int32)
    return lax.gather(vec, idx, _GDN, slice_sizes=(1,),
                      mode=lax.GatherScatterMode.PROMISE_IN_BOUNDS)


COUNT = NCHUNKS // NS
EXTRA = NCHUNKS - COUNT * NS


def _propagate(pk, cur):
    @functools.partial(
        pl.kernel,
        out_type=jax.ShapeDtypeStruct((N, D), jnp.float32),
        mesh=_mesh,
        scratch_types=[
            pltpu.VMEM_SHARED((HALF, D), jnp.float32),
            pltpu.VMEM((ZROWS, D), jnp.float32),
            pltpu.VMEM((2, 3, C), jnp.int32),
            pltpu.VMEM((2, C), jnp.int32),
            pltpu.VMEM((2, C, D), jnp.float32),
            pltpu.SemaphoreType.DMA,
            pltpu.SemaphoreType.DMA,
            pltpu.SemaphoreType.DMA,
            pltpu.SemaphoreType.DMA,
            pltpu.SemaphoreType.DMA,
            pltpu.SemaphoreType.DMA,
        ],
        compiler_params=pltpu.CompilerParams(use_tc_tiling_on_sc=False, needs_layout_passes=False),
    )
    def k(pk_hbm, cur_hbm, out_hbm,
          accum, zbuf, ebuf, ldstb, rows,
          sem_e0, sem_e1, sem_g0, sem_g1, sem_s0, sem_s1):
        cid = lax.axis_index("c")
        sid = lax.axis_index("s")
        sem_e = (sem_e0, sem_e1)
        sem_g = (sem_g0, sem_g1)
        sem_s = (sem_s0, sem_s1)
        zero16 = jnp.zeros((16,), jnp.float32)
        lo = cid * HALF

        def kchunk(i):
            return sid + i * NS

        def issue_ecopy(i, p):
            kc = jnp.minimum(kchunk(i), NCHUNKS - 1)
            pltpu.async_copy(pk_hbm.at[kc], ebuf.at[p], sem_e[p])

        def wait_ecopy(p):
            pltpu.make_async_copy(pk_hbm.at[0], ebuf.at[p], sem_e[p]).wait()

        def issue_gather(p):
            pltpu.async_copy(cur_hbm.at[ebuf.at[p, 1]], rows.at[p], sem_g[p])

        def wait_gather(p):
            pltpu.make_async_copy(cur_hbm.at[ebuf.at[p, 1]], rows.at[p],
                                  sem_g[p]).wait()

        def issue_scatter(p):
            pltpu.async_copy(rows.at[p], accum.at[ldstb.at[p]], sem_s[p],
                             add=True)

        def wait_scatter(p):
            pltpu.make_async_copy(rows.at[p], accum.at[ldstb.at[p]],
                                  sem_s[p]).wait()

        def process(p):
            for v in range(C // 16):
                dv = ebuf[p, 0, pl.ds(v * 16, 16)]
                wv = lax.bitcast_convert_type(ebuf[p, 2, pl.ds(v * 16, 16)],
                                              jnp.float32)
                m = (dv >= lo) & (dv < lo + HALF)
                ldstb[p, pl.ds(v * 16, 16)] = jnp.where(m, dv - lo, 0)
                wg = jnp.where(m, wv, jnp.zeros((16,), jnp.float32))
                for t in range(16):
                    j = v * 16 + t
                    ws = _lane_bcast(wg, t)
                    rows[p, j, pl.ds(0, 16)] = rows[p, j, pl.ds(0, 16)] * ws
                    rows[p, j, pl.ds(16, 16)] = rows[p, j, pl.ds(16, 16)] * ws

        def zb(i, carry):
            zbuf[i, pl.ds(0, 16)] = zero16
            zbuf[i, pl.ds(16, 16)] = zero16
            return carry
        lax.fori_loop(0, ZROWS, zb, 0)

        nblocks_mine = (NBLOCKS - sid + NS - 1) // NS

        def zero_block(i, carry):
            pltpu.sync_copy(zbuf, accum.at[pl.ds((sid + i * NS) * ZROWS, ZROWS)])
            return carry
        lax.fori_loop(0, nblocks_mine, zero_block, 0)

        issue_ecopy(0, 0)
        wait_ecopy(0)
        issue_gather(0)
        issue_ecopy(1, 1)
        plsc.subcore_barrier()

        def halfstep(i, p, scatter_wait_cond=None):
            wait_gather(1 - p)
            process(1 - p)
            issue_scatter(1 - p)
            wait_ecopy(p)
            if scatter_wait_cond is None:
                wait_scatter(p)
            else:
                @pl.when(scatter_wait_cond)
                def _():
                    wait_scatter(p)
            issue_gather(p)
            issue_ecopy(i + 1, 1 - p)

        def pair(jj, carry):
            i1 = 2 * jj + 1
            halfstep(i1, 1, scatter_wait_cond=jj > 0)
            halfstep(i1 + 1, 0)
            return carry
        lax.fori_loop(0, (COUNT - 1) // 2, pair, 0)

        wait_gather(0)
        process(0)
        issue_scatter(0)
        wait_ecopy(1)
        wait_scatter(1)

        @pl.when(sid < EXTRA)
        def _():
            issue_gather(1)
            wait_gather(1)
            process(1)
            pltpu.sync_copy(rows.at[1], accum.at[ldstb.at[1]], add=True)

        wait_scatter(0)
        plsc.subcore_barrier()

        def write_block(i, carry):
            r0 = (sid + i * NS) * ZROWS
            pltpu.sync_copy(accum.at[pl.ds(r0, ZROWS)],
                            out_hbm.at[pl.ds(cid * HALF + r0, ZROWS)])
            return carry
        lax.fori_loop(0, nblocks_mine, write_block, 0)

    return k(pk, cur)


def _score(pid, hid, t0, t1, t2, t3):
    @functools.partial(
        pl.kernel,
        out_type=jax.ShapeDtypeStruct((B,), jnp.float32),
        mesh=_mesh,
        scratch_types=[
            pltpu.VMEM((PP,), jnp.int32),
            pltpu.VMEM((PP,), jnp.int32),
            pltpu.VMEM((PP, D), jnp.float32),
            pltpu.VMEM((PP, D), jnp.float32),
            pltpu.VMEM((PP, D), jnp.float32),
            pltpu.VMEM((PP,), jnp.float32),
            pltpu.SemaphoreType.DMA,
        ],
        compiler_params=pltpu.CompilerParams(use_tc_tiling_on_sc=False, needs_layout_passes=False),
    )
    def k(pid_hbm, hid_hbm, t0_hbm, t1_hbm, t2_hbm, t3_hbm, out_hbm,
          pidb, hidb, pacc, hacc, tmp, sb, sem):
        cid = lax.axis_index("c")
        sid = lax.axis_index("s")
        base = (cid * NS + sid) * PP
        pltpu.sync_copy(pid_hbm.at[pl.ds(base, PP)], pidb)
        pltpu.sync_copy(hid_hbm.at[pl.ds(base, PP)], hidb)

        def off(i, carry):
            hidb[pl.ds(i * 16, 16)] = hidb[pl.ds(i * 16, 16)] + NUM_P
            return carry
        lax.fori_loop(0, PP // 16, off, 0)

        def accumulate(idxb, acc):
            pltpu.async_copy(t0_hbm.at[idxb], acc, sem).wait()
            for t_hbm in (t1_hbm, t2_hbm, t3_hbm):
                pltpu.async_copy(t_hbm.at[idxb], tmp, sem).wait()

                def addv(r, carry):
                    acc[r, pl.ds(0, 16)] = acc[r, pl.ds(0, 16)] + tmp[r, pl.ds(0, 16)]
                    acc[r, pl.ds(16, 16)] = acc[r, pl.ds(16, 16)] + tmp[r, pl.ds(16, 16)]
                    return carry
                lax.fori_loop(0, PP, addv, 0)

        accumulate(pidb, pacc)
        accumulate(hidb, hacc)

        iota = lax.iota(jnp.int32, 16)

        def dotg(g, carry):
            rowidx = g * 16 + iota
            acc = jnp.zeros((16,), jnp.float32)
            for d in range(D):
                col = jnp.full((16,), d, jnp.int32)
                pc = plsc.load_gather(pacc, [rowidx, col])
                hc = plsc.load_gather(hacc, [rowidx, col])
                acc = acc + pc * hc
            sb[pl.ds(g * 16, 16)] = acc * jnp.float32(1.0 / 16.0)
            return carry
        lax.fori_loop(0, PP // 16, dotg, 0)
        pltpu.sync_copy(sb, out_hbm.at[pl.ds(base, PP)])

    return k(pid, hid, t0, t1, t2, t3)


def kernel(person_ids, hobby_ids, edge_index, edge_weight, person_emb, hobby_emb):
    dst = edge_index[0].astype(jnp.int32)
    src = edge_index[1].astype(jnp.int32)
    w = edge_weight.astype(jnp.float32)
    pk = jnp.stack([dst.reshape(NCHUNKS, C),
                    src.reshape(NCHUNKS, C),
                    lax.bitcast_convert_type(w, jnp.int32).reshape(NCHUNKS, C)],
                   axis=1)
    t0 = jnp.concatenate([person_emb, hobby_emb], axis=0)
    t1 = _propagate(pk, t0)
    t2 = _propagate(pk, t1)
    t3 = _propagate(pk, t2)
    return _score(person_ids.astype(jnp.int32), hobby_ids.astype(jnp.int32),
                  t0, t1, t2, t3)


# speedup vs baseline: 14.9432x; 1.7931x over previous
import functools

import jax
import jax.numpy as jnp
from jax import lax
from jax.experimental import pallas as pl
from jax.experimental.pallas import tpu as pltpu
from jax.experimental.pallas import tpu_sc as plsc

NUM_P = 60000
NUM_H = 40000
N = 100000
D = 32
E = 1600000
B = 4096
C = 128
NCHUNKS = E // C
NC = 2
NS = 16
NW = NC * NS
HALF = N // NC
ZROWS = 400
NBLOCKS = HALF // ZROWS
PP = B // NW

CNTU = (NCHUNKS + NW - 1) // NW
CAPC = 12560
CAPE = CAPC * C
STAGE = 1280
FLUSH = 1024

_mesh = plsc.VectorSubcoreMesh(core_axis_name="c", subcore_axis_name="s")
_params = pltpu.CompilerParams(use_tc_tiling_on_sc=False,
                               needs_layout_passes=False)

_GDN = lax.GatherDimensionNumbers(
    offset_dims=(), collapsed_slice_dims=(0,), start_index_map=(0,))


def _lane_bcast(vec, t):
    idx = jnp.full((16, 1), t, jnp.int32)
    return lax.gather(vec, idx, _GDN, slice_sizes=(1,),
                      mode=lax.GatherScatterMode.PROMISE_IN_BOUNDS)


def _splat(x):
    return jnp.full((16,), x, jnp.int32)


def _count(pk):
    @functools.partial(
        pl.kernel,
        out_type=jax.ShapeDtypeStruct((2, NW, 16), jnp.int32),
        mesh=_mesh,
        scratch_types=[
            pltpu.VMEM((2, 3, C), jnp.int32),
            pltpu.VMEM((16,), jnp.int32),
            pltpu.SemaphoreType.DMA,
            pltpu.SemaphoreType.DMA,
        ],
        compiler_params=_params,
    )
    def k(pk_hbm, counts_hbm, ebuf, cbuf, sem_e0, sem_e1):
        cid = lax.axis_index("c")
        sid = lax.axis_index("s")
        me = cid * NS + sid
        sem_e = (sem_e0, sem_e1)

        def issue_e(i, p):
            kc = jnp.minimum(me + i * NW, NCHUNKS - 1)
            pltpu.async_copy(pk_hbm.at[kc], ebuf.at[p], sem_e[p])

        def wait_e(p):
            pltpu.make_async_copy(pk_hbm.at[0], ebuf.at[p], sem_e[p]).wait()

        def process(i, p, acc):
            valid = (me + i * NW < NCHUNKS).astype(jnp.int32)
            vv = _splat(valid)
            for v in range(C // 16):
                dv = ebuf[p, 0, pl.ds(v * 16, 16)]
                acc = acc + jnp.where(dv < HALF, vv, _splat(0))
            return acc

        issue_e(0, 0)
        issue_e(1, 1)
        acc0 = jnp.zeros((16,), jnp.int32)
        wait_e(0)
        acc0 = process(0, 0, acc0)
        issue_e(2, 0)

        def pair(jj, acc):
            i1 = 2 * jj + 1
            wait_e(1)
            acc = process(i1, 1, acc)
            issue_e(i1 + 2, 1)
            wait_e(0)
            acc = process(i1 + 1, 0, acc)
            issue_e(i1 + 3, 0)
            return acc
        acc0 = lax.fori_loop(0, (CNTU - 1) // 2, pair, acc0)
        wait_e(1)
        wait_e(0)

        nlo = jnp.sum(acc0)
        nvalid = (NCHUNKS - me + NW - 1) // NW
        nhi = C * nvalid - nlo
        clo = (nlo + C - 1) // C
        chi = (nhi + C - 1) // C
        cbuf[pl.ds(0, 16)] = _splat(clo)
        pltpu.sync_copy(cbuf, counts_hbm.at[0, me])
        cbuf[pl.ds(0, 16)] = _splat(chi)
        pltpu.sync_copy(cbuf, counts_hbm.at[1, me])

    return k(pk)


def _odd_ceil_chunks(tot):
    return ((tot + NS - 1) // NS) | 1


def _compact(pk, counts):
    out = jax.ShapeDtypeStruct((2, CAPE), jnp.int32)

    @functools.partial(
        pl.kernel,
        out_type=(out, out, out),
        mesh=_mesh,
        scratch_types=[
            pltpu.VMEM((2, 3, C), jnp.int32),
            pltpu.VMEM((2, NW, 16), jnp.int32),
            pltpu.VMEM((6, STAGE), jnp.int32),
            pltpu.VMEM((C,), jnp.int32),
            pltpu.SemaphoreType.DMA,
            pltpu.SemaphoreType.DMA,
        ],
        compiler_params=_params,
    )
    def k(pk_hbm, counts_hbm, fd_hbm, fs_hbm, fw_hbm,
          ebuf, cbuf, st, zc, sem_e0, sem_e1):
        cid = lax.axis_index("c")
        sid = lax.axis_index("s")
        me = cid * NS + sid
        sem_e = (sem_e0, sem_e1)
        outs = (fd_hbm, fs_hbm, fw_hbm)

        pltpu.sync_copy(counts_hbm, cbuf)
        zero16 = jnp.zeros((16,), jnp.int32)
        blo = zero16
        bhi = zero16
        tlo = zero16
        thi = zero16
        for t in range(NW):
            clv = cbuf[0, t, pl.ds(0, 16)]
            chv = cbuf[1, t, pl.ds(0, 16)]
            pred = _splat((t < me).astype(jnp.int32)) > 0
            blo = blo + jnp.where(pred, clv, zero16)
            bhi = bhi + jnp.where(pred, chv, zero16)
            tlo = tlo + clv
            thi = thi + chv
        base = (jnp.max(blo) * C, jnp.max(bhi) * C)
        tot = (jnp.max(tlo), jnp.max(thi))

        for j in range(C // 16):
            zc[pl.ds(j * 16, 16)] = zero16

        def issue_e(i, p):
            kc = jnp.minimum(me + i * NW, NCHUNKS - 1)
            pltpu.async_copy(pk_hbm.at[kc], ebuf.at[p], sem_e[p])

        def wait_e(p):
            pltpu.make_async_copy(pk_hbm.at[0], ebuf.at[p], sem_e[p]).wait()

        def process(i, p, carry):
            ptr_lo, ptr_hi, wp_lo, wp_hi = carry
            valid = (me + i * NW < NCHUNKS).astype(jnp.int32)
            vmask = _splat(valid) > 0
            for v in range(C // 16):
                dv = ebuf[p, 0, pl.ds(v * 16, 16)]
                sv = ebuf[p, 1, pl.ds(v * 16, 16)]
                wv = ebuf[p, 2, pl.ds(v * 16, 16)]
                mlo = (dv < HALF) & vmask
                mhi = (dv >= HALF) & vmask
                nlo = jnp.sum(jnp.where(mlo, _splat(1), zero16))
                nhi = valid * 16 - nlo
                plsc.store_compressed(st.at[0, pl.ds(ptr_lo, 16)], dv, mask=mlo)
                plsc.store_compressed(st.at[1, pl.ds(ptr_lo, 16)], sv, mask=mlo)
                plsc.store_compressed(st.at[2, pl.ds(ptr_lo, 16)], wv, mask=mlo)
                plsc.store_compressed(st.at[3, pl.ds(ptr_hi, 16)], dv - HALF, mask=mhi)
                plsc.store_compressed(st.at[4, pl.ds(ptr_hi, 16)], sv, mask=mhi)
                plsc.store_compressed(st.at[5, pl.ds(ptr_hi, 16)], wv, mask=mhi)
                ptr_lo = ptr_lo + nlo
                ptr_hi = ptr_hi + nhi
            for h, ptr, wp, f0 in ((0, ptr_lo, wp_lo, 0), (1, ptr_hi, wp_hi, 3)):
                do = ptr >= FLUSH

                @pl.when(do)
                def _(h=h, wp=wp, f0=f0):
                    off = pl.multiple_of(base[h] + wp, 8)
                    for f in range(3):
                        pltpu.sync_copy(st.at[f0 + f, pl.ds(0, FLUSH)],
                                        outs[f].at[h, pl.ds(off, FLUSH)])
                    for f in range(3):
                        for j in range(10):
                            st[f0 + f, pl.ds(j * 16, 16)] = (
                                st[f0 + f, pl.ds(FLUSH + j * 16, 16)])
                if h == 0:
                    ptr_lo = jnp.where(do, ptr_lo - FLUSH, ptr_lo)
                    wp_lo = jnp.where(do, wp_lo + FLUSH, wp_lo)
                else:
                    ptr_hi = jnp.where(do, ptr_hi - FLUSH, ptr_hi)
                    wp_hi = jnp.where(do, wp_hi + FLUSH, wp_hi)
            return (ptr_lo, ptr_hi, wp_lo, wp_hi)

        issue_e(0, 0)
        issue_e(1, 1)
        carry = (jnp.int32(0), jnp.int32(0), jnp.int32(0), jnp.int32(0))
        wait_e(0)
        carry = process(0, 0, carry)
        issue_e(2, 0)

        def pair(jj, carry):
            i1 = 2 * jj + 1
            wait_e(1)
            carry = process(i1, 1, carry)
            issue_e(i1 + 2, 1)
            wait_e(0)
            carry = process(i1 + 1, 0, carry)
            issue_e(i1 + 3, 0)
            return carry
        carry = lax.fori_loop(0, (CNTU - 1) // 2, pair, carry)
        wait_e(1)
        wait_e(0)
        ptr_lo, ptr_hi, wp_lo, wp_hi = carry

        lanes = lax.iota(jnp.int32, 16)
        for ptr, wp, h, f0 in ((ptr_lo, wp_lo, 0, 0), (ptr_hi, wp_hi, 1, 3)):
            start16 = ptr & ~15
            keep = lanes < (ptr - start16)
            for f in range(3):
                vcur = st[f0 + f, pl.ds(start16, 16)]
                st[f0 + f, pl.ds(start16, 16)] = jnp.where(keep, vcur, zero16)
                for j in range(1, 8):
                    st[f0 + f, pl.ds(start16 + j * 16, 16)] = zero16
            nrem = (ptr + C - 1) // C

            def dflush(j, carry2, wp=wp, h=h, f0=f0):
                off = pl.multiple_of(base[h] + wp + j * C, 8)
                for f in range(3):
                    pltpu.sync_copy(
                        st.at[f0 + f, pl.ds(j * C, C)],
                        outs[f].at[h, pl.ds(off, C)])
                return carry2
            lax.fori_loop(0, nrem, dflush, 0)

        for h in range(2):
            m = _odd_ceil_chunks(tot[h])
            npad = m * NS - tot[h]

            def pchunk(j, carry2, h=h, npad=npad, m=m):
                pc = pl.multiple_of((tot[h] + me + j * NW) * C, 8)
                for f in range(3):
                    pltpu.sync_copy(zc, outs[f].at[h, pl.ds(pc, C)])
                return carry2
            npad_mine = jnp.maximum((npad - me + NW - 1) // NW, 0)
            lax.fori_loop(0, npad_mine, pchunk, 0)

    return k(pk, counts)


def _propagate(fd, fs, fw, counts, cur):
    @functools.partial(
        pl.kernel,
        out_type=jax.ShapeDtypeStruct((N, D), jnp.float32),
        mesh=_mesh,
        scratch_types=[
            pltpu.VMEM_SHARED((HALF, D), jnp.float32),
            pltpu.VMEM((ZROWS, D), jnp.float32),
            pltpu.VMEM((2, 3, C), jnp.int32),
            pltpu.VMEM((2, NW, 16), jnp.int32),
            pltpu.VMEM((2, C, D), jnp.float32),
            pltpu.SemaphoreType.DMA,
            pltpu.SemaphoreType.DMA,
            pltpu.SemaphoreType.DMA,
            pltpu.SemaphoreType.DMA,
            pltpu.SemaphoreType.DMA,
            pltpu.SemaphoreType.DMA,
        ],
        compiler_params=_params,
    )
    def k(fd_hbm, fs_hbm, fw_hbm, counts_hbm, cur_hbm, out_hbm,
          accum, zbuf, ebuf, cbuf, rows,
          sem_e0, sem_e1, sem_g0, sem_g1, sem_s0, sem_s1):
        cid = lax.axis_index("c")
        sid = lax.axis_index("s")
        sem_e = (sem_e0, sem_e1)
        sem_g = (sem_g0, sem_g1)
        sem_s = (sem_s0, sem_s1)
        zero16 = jnp.zeros((16,), jnp.float32)

        pltpu.sync_copy(counts_hbm, cbuf)
        tot = jnp.zeros((16,), jnp.int32)
        for t in range(NW):
            tot = tot + cbuf[cid, t, pl.ds(0, 16)]
        m = _odd_ceil_chunks(jnp.max(tot))

        def echunk(i):
            return pl.multiple_of(jnp.minimum(sid + i * NS, CAPC - 1) * C, 8)

        def issue_ecopy(i, p):
            eb = echunk(i)
            pltpu.async_copy(fd_hbm.at[cid, pl.ds(eb, C)], ebuf.at[p, 0], sem_e[p])
            pltpu.async_copy(fs_hbm.at[cid, pl.ds(eb, C)], ebuf.at[p, 1], sem_e[p])
            pltpu.async_copy(fw_hbm.at[cid, pl.ds(eb, C)], ebuf.at[p, 2], sem_e[p])

        def wait_ecopy(p):
            for j in range(3):
                pltpu.make_async_copy(fd_hbm.at[0, pl.ds(0, C)],
                                      ebuf.at[p, j], sem_e[p]).wait()

        def issue_gather(p):
            pltpu.async_copy(cur_hbm.at[ebuf.at[p, 1]], rows.at[p], sem_g[p])

        def wait_gather(p):
            pltpu.make_async_copy(cur_hbm.at[ebuf.at[p, 1]], rows.at[p],
                                  sem_g[p]).wait()

        def issue_scatter(p):
            pltpu.async_copy(rows.at[p], accum.at[ebuf.at[p, 0]], sem_s[p],
                             add=True)

        def wait_scatter(p):
            pltpu.make_async_copy(rows.at[p], accum.at[ebuf.at[p, 0]],
                                  sem_s[p]).wait()

        def process(p):
            for v in range(C // 16):
                wg = lax.bitcast_convert_type(ebuf[p, 2, pl.ds(v * 16, 16)],
                                              jnp.float32)
                for t in range(16):
                    j = v * 16 + t
                    ws = _lane_bcast(wg, t)
                    rows[p, j, pl.ds(0, 16)] = rows[p, j, pl.ds(0, 16)] * ws
                    rows[p, j, pl.ds(16, 16)] = rows[p, j, pl.ds(16, 16)] * ws

        def zb(i, carry):
            zbuf[i, pl.ds(0, 16)] = zero16
            zbuf[i, pl.ds(16, 16)] = zero16
            return carry
        lax.fori_loop(0, ZROWS, zb, 0)

        nblocks_mine = (NBLOCKS - sid + NS - 1) // NS

        def zero_block(i, carry):
            pltpu.sync_copy(zbuf, accum.at[pl.ds((sid + i * NS) * ZROWS, ZROWS)])
            return carry
        lax.fori_loop(0, nblocks_mine, zero_block, 0)

        issue_ecopy(0, 0)
        wait_ecopy(0)
        issue_gather(0)
        issue_ecopy(1, 1)
        plsc.subcore_barrier()

        def halfstep(i, p, scatter_wait_cond=None):
            wait_gather(1 - p)
            process(1 - p)
            issue_scatter(1 - p)
            wait_ecopy(p)
            if scatter_wait_cond is None:
                wait_scatter(p)
            else:
                @pl.when(scatter_wait_cond)
                def _():
                    wait_scatter(p)
            issue_gather(p)
            issue_ecopy(i + 1, 1 - p)

        def pair(jj, carry):
            i1 = 2 * jj + 1
            halfstep(i1, 1, scatter_wait_cond=jj > 0)
            halfstep(i1 + 1, 0)
            return carry
        lax.fori_loop(0, (m - 1) // 2, pair, 0)

        wait_gather(0)
        process(0)
        issue_scatter(0)
        wait_ecopy(1)

        @pl.when(m > 1)
        def _():
            wait_scatter(1)

        wait_scatter(0)
        plsc.subcore_barrier()

        def write_block(i, carry):
            r0 = (sid + i * NS) * ZROWS
            pltpu.sync_copy(accum.at[pl.ds(r0, ZROWS)],
                            out_hbm.at[pl.ds(cid * HALF + r0, ZROWS)])
            return carry
        lax.fori_loop(0, nblocks_mine, write_block, 0)

    return k(fd, fs, fw, counts, cur)


def _score(pid, hid, t0, t1, t2, t3):
    @functools.partial(
        pl.kernel,
        out_type=jax.ShapeDtypeStruct((B,), jnp.float32),
        mesh=_mesh,
        scratch_types=[
            pltpu.VMEM((PP,), jnp.int32),
            pltpu.VMEM((PP,), jnp.int32),
            pltpu.VMEM((PP, D), jnp.float32),
            pltpu.VMEM((PP, D), jnp.float32),
            pltpu.VMEM((PP, D), jnp.float32),
            pltpu.VMEM((PP,), jnp.float32),
            pltpu.SemaphoreType.DMA,
        ],
        compiler_params=_params,
    )
    def k(pid_hbm, hid_hbm, t0_hbm, t1_hbm, t2_hbm, t3_hbm, out_hbm,
          pidb, hidb, pacc, hacc, tmp, sb, sem):
        cid = lax.axis_index("c")
        sid = lax.axis_index("s")
        base = (cid * NS + sid) * PP
        pltpu.sync_copy(pid_hbm.at[pl.ds(base, PP)], pidb)
        pltpu.sync_copy(hid_hbm.at[pl.ds(base, PP)], hidb)

        def off(i, carry):
            hidb[pl.ds(i * 16, 16)] = hidb[pl.ds(i * 16, 16)] + NUM_P
            return carry
        lax.fori_loop(0, PP // 16, off, 0)

        def accumulate(idxb, acc):
            pltpu.async_copy(t0_hbm.at[idxb], acc, sem).wait()
            for t_hbm in (t1_hbm, t2_hbm, t3_hbm):
                pltpu.async_copy(t_hbm.at[idxb], tmp, sem).wait()

                def addv(r, carry):
                    acc[r, pl.ds(0, 16)] = acc[r, pl.ds(0, 16)] + tmp[r, pl.ds(0, 16)]
                    acc[r, pl.ds(16, 16)] = acc[r, pl.ds(16, 16)] + tmp[r, pl.ds(16, 16)]
                    return carry
                lax.fori_loop(0, PP, addv, 0)

        accumulate(pidb, pacc)
        accumulate(hidb, hacc)

        iota = lax.iota(jnp.int32, 16)

        def dotg(g, carry):
            rowidx = g * 16 + iota
            acc = jnp.zeros((16,), jnp.float32)
            for d in range(D):
                col = jnp.full((16,), d, jnp.int32)
                pc = plsc.load_gather(pacc, [rowidx, col])
                hc = plsc.load_gather(hacc, [rowidx, col])
                acc = acc + pc * hc
            sb[pl.ds(g * 16, 16)] = acc * jnp.float32(1.0 / 16.0)
            return carry
        lax.fori_loop(0, PP // 16, dotg, 0)
        pltpu.sync_copy(sb, out_hbm.at[pl.ds(base, PP)])

    return k(pid, hid, t0, t1, t2, t3)


def kernel(person_ids, hobby_ids, edge_index, edge_weight, person_emb, hobby_emb):
    dst = edge_index[0].astype(jnp.int32)
    src = edge_index[1].astype(jnp.int32)
    w = edge_weight.astype(jnp.float32)
    pk = jnp.stack([dst.reshape(NCHUNKS, C),
                    src.reshape(NCHUNKS, C),
                    lax.bitcast_convert_type(w, jnp.int32).reshape(NCHUNKS, C)],
                   axis=1)
    counts = _count(pk)
    fd, fs, fw = _compact(pk, counts)
    t0 = jnp.concatenate([person_emb, hobby_emb], axis=0)
    t1 = _propagate(fd, fs, fw, counts, t0)
    t2 = _propagate(fd, fs, fw, counts, t1)
    t3 = _propagate(fd, fs, fw, counts, t2)
    return _score(person_ids.astype(jnp.int32), hobby_ids.astype(jnp.int32),
                  t0, t1, t2, t3)


# speedup vs baseline: 16.9721x; 1.1358x over previous
import functools

import jax
import jax.numpy as jnp
from jax import lax
from jax.experimental import pallas as pl
from jax.experimental.pallas import tpu as pltpu
from jax.experimental.pallas import tpu_sc as plsc

NUM_P = 60000
NUM_H = 40000
N = 100000
D = 32
E = 1600000
B = 4096
C = 128
NCHUNKS = E // C
NC = 2
NS = 16
NW = NC * NS
HALF = N // NC
ZROWS = 200
NBLOCKS = HALF // ZROWS
PP = B // NW

CNTU = (NCHUNKS + NW - 1) // NW
CAPC = 12576
CAPE = CAPC * C
STAGE = 1280
FLUSH = 1024

_mesh = plsc.VectorSubcoreMesh(core_axis_name="c", subcore_axis_name="s")
_params = pltpu.CompilerParams(use_tc_tiling_on_sc=False,
                               needs_layout_passes=False)

_GDN = lax.GatherDimensionNumbers(
    offset_dims=(), collapsed_slice_dims=(0,), start_index_map=(0,))


def _lane_bcast(vec, t):
    idx = jnp.full((16, 1), t, jnp.int32)
    return lax.gather(vec, idx, _GDN, slice_sizes=(1,),
                      mode=lax.GatherScatterMode.PROMISE_IN_BOUNDS)


def _splat(x):
    return jnp.full((16,), x, jnp.int32)


def _count(pk):
    @functools.partial(
        pl.kernel,
        out_type=jax.ShapeDtypeStruct((2, NW, 16), jnp.int32),
        mesh=_mesh,
        scratch_types=[
            pltpu.VMEM((2, 3, C), jnp.int32),
            pltpu.VMEM((16,), jnp.int32),
            pltpu.SemaphoreType.DMA,
            pltpu.SemaphoreType.DMA,
        ],
        compiler_params=_params,
    )
    def k(pk_hbm, counts_hbm, ebuf, cbuf, sem_e0, sem_e1):
        cid = lax.axis_index("c")
        sid = lax.axis_index("s")
        me = cid * NS + sid
        sem_e = (sem_e0, sem_e1)

        def issue_e(i, p):
            kc = jnp.minimum(me + i * NW, NCHUNKS - 1)
            pltpu.async_copy(pk_hbm.at[kc], ebuf.at[p], sem_e[p])

        def wait_e(p):
            pltpu.make_async_copy(pk_hbm.at[0], ebuf.at[p], sem_e[p]).wait()

        def process(i, p, acc):
            valid = (me + i * NW < NCHUNKS).astype(jnp.int32)
            vv = _splat(valid)
            for v in range(C // 16):
                dv = ebuf[p, 0, pl.ds(v * 16, 16)]
                acc = acc + jnp.where(dv < HALF, vv, _splat(0))
            return acc

        issue_e(0, 0)
        issue_e(1, 1)
        acc0 = jnp.zeros((16,), jnp.int32)
        wait_e(0)
        acc0 = process(0, 0, acc0)
        issue_e(2, 0)

        def pair(jj, acc):
            i1 = 2 * jj + 1
            wait_e(1)
            acc = process(i1, 1, acc)
            issue_e(i1 + 2, 1)
            wait_e(0)
            acc = process(i1 + 1, 0, acc)
            issue_e(i1 + 3, 0)
            return acc
        acc0 = lax.fori_loop(0, (CNTU - 1) // 2, pair, acc0)
        wait_e(1)
        wait_e(0)

        nlo = jnp.sum(acc0)
        nvalid = (NCHUNKS - me + NW - 1) // NW
        nhi = C * nvalid - nlo
        clo = (nlo + C - 1) // C
        chi = (nhi + C - 1) // C
        cbuf[pl.ds(0, 16)] = _splat(clo)
        pltpu.sync_copy(cbuf, counts_hbm.at[0, me])
        cbuf[pl.ds(0, 16)] = _splat(chi)
        pltpu.sync_copy(cbuf, counts_hbm.at[1, me])

    return k(pk)


def _odd_steps(tot):
    return ((tot + 2 * NS - 1) // (2 * NS)) | 1


def _compact(pk, counts):
    out = jax.ShapeDtypeStruct((2, CAPE), jnp.int32)

    @functools.partial(
        pl.kernel,
        out_type=(out, out, out),
        mesh=_mesh,
        scratch_types=[
            pltpu.VMEM((2, 3, C), jnp.int32),
            pltpu.VMEM((2, NW, 16), jnp.int32),
            pltpu.VMEM((6, STAGE), jnp.int32),
            pltpu.VMEM((C,), jnp.int32),
            pltpu.SemaphoreType.DMA,
            pltpu.SemaphoreType.DMA,
        ],
        compiler_params=_params,
    )
    def k(pk_hbm, counts_hbm, fd_hbm, fs_hbm, fw_hbm,
          ebuf, cbuf, st, zc, sem_e0, sem_e1):
        cid = lax.axis_index("c")
        sid = lax.axis_index("s")
        me = cid * NS + sid
        sem_e = (sem_e0, sem_e1)
        outs = (fd_hbm, fs_hbm, fw_hbm)

        pltpu.sync_copy(counts_hbm, cbuf)
        zero16 = jnp.zeros((16,), jnp.int32)
        blo = zero16
        bhi = zero16
        tlo = zero16
        thi = zero16
        for t in range(NW):
            clv = cbuf[0, t, pl.ds(0, 16)]
            chv = cbuf[1, t, pl.ds(0, 16)]
            pred = _splat((t < me).astype(jnp.int32)) > 0
            blo = blo + jnp.where(pred, clv, zero16)
            bhi = bhi + jnp.where(pred, chv, zero16)
            tlo = tlo + clv
            thi = thi + chv
        base = (jnp.max(blo) * C, jnp.max(bhi) * C)
        tot = (jnp.max(tlo), jnp.max(thi))

        for j in range(C // 16):
            zc[pl.ds(j * 16, 16)] = zero16

        def issue_e(i, p):
            kc = jnp.minimum(me + i * NW, NCHUNKS - 1)
            pltpu.async_copy(pk_hbm.at[kc], ebuf.at[p], sem_e[p])

        def wait_e(p):
            pltpu.make_async_copy(pk_hbm.at[0], ebuf.at[p], sem_e[p]).wait()

        def process(i, p, carry):
            ptr_lo, ptr_hi, wp_lo, wp_hi = carry
            valid = (me + i * NW < NCHUNKS).astype(jnp.int32)
            vmask = _splat(valid) > 0
            for v in range(C // 16):
                dv = ebuf[p, 0, pl.ds(v * 16, 16)]
                sv = ebuf[p, 1, pl.ds(v * 16, 16)]
                wv = ebuf[p, 2, pl.ds(v * 16, 16)]
                mlo = (dv < HALF) & vmask
                mhi = (dv >= HALF) & vmask
                nlo = jnp.sum(jnp.where(mlo, _splat(1), zero16))
                nhi = valid * 16 - nlo
                plsc.store_compressed(st.at[0, pl.ds(ptr_lo, 16)], dv, mask=mlo)
                plsc.store_compressed(st.at[1, pl.ds(ptr_lo, 16)], sv, mask=mlo)
                plsc.store_compressed(st.at[2, pl.ds(ptr_lo, 16)], wv, mask=mlo)
                plsc.store_compressed(st.at[3, pl.ds(ptr_hi, 16)], dv - HALF, mask=mhi)
                plsc.store_compressed(st.at[4, pl.ds(ptr_hi, 16)], sv, mask=mhi)
                plsc.store_compressed(st.at[5, pl.ds(ptr_hi, 16)], wv, mask=mhi)
                ptr_lo = ptr_lo + nlo
                ptr_hi = ptr_hi + nhi
            for h, ptr, wp, f0 in ((0, ptr_lo, wp_lo, 0), (1, ptr_hi, wp_hi, 3)):
                do = ptr >= FLUSH

                @pl.when(do)
                def _(h=h, wp=wp, f0=f0):
                    off = pl.multiple_of(base[h] + wp, 8)
                    for f in range(3):
                        pltpu.sync_copy(st.at[f0 + f, pl.ds(0, FLUSH)],
                                        outs[f].at[h, pl.ds(off, FLUSH)])
                    for f in range(3):
                        for j in range(10):
                            st[f0 + f, pl.ds(j * 16, 16)] = (
                                st[f0 + f, pl.ds(FLUSH + j * 16, 16)])
                if h == 0:
                    ptr_lo = jnp.where(do, ptr_lo - FLUSH, ptr_lo)
                    wp_lo = jnp.where(do, wp_lo + FLUSH, wp_lo)
                else:
                    ptr_hi = jnp.where(do, ptr_hi - FLUSH, ptr_hi)
                    wp_hi = jnp.where(do, wp_hi + FLUSH, wp_hi)
            return (ptr_lo, ptr_hi, wp_lo, wp_hi)

        issue_e(0, 0)
        issue_e(1, 1)
        carry = (jnp.int32(0), jnp.int32(0), jnp.int32(0), jnp.int32(0))
        wait_e(0)
        carry = process(0, 0, carry)
        issue_e(2, 0)

        def pair(jj, carry):
            i1 = 2 * jj + 1
            wait_e(1)
            carry = process(i1, 1, carry)
            issue_e(i1 + 2, 1)
            wait_e(0)
            carry = process(i1 + 1, 0, carry)
            issue_e(i1 + 3, 0)
            return carry
        carry = lax.fori_loop(0, (CNTU - 1) // 2, pair, carry)
        wait_e(1)
        wait_e(0)
        ptr_lo, ptr_hi, wp_lo, wp_hi = carry

        lanes = lax.iota(jnp.int32, 16)
        for ptr, wp, h, f0 in ((ptr_lo, wp_lo, 0, 0), (ptr_hi, wp_hi, 1, 3)):
            start16 = ptr & ~15
            keep = lanes < (ptr - start16)
            for f in range(3):
                vcur = st[f0 + f, pl.ds(start16, 16)]
                st[f0 + f, pl.ds(start16, 16)] = jnp.where(keep, vcur, zero16)
                for j in range(1, 8):
                    st[f0 + f, pl.ds(start16 + j * 16, 16)] = zero16
            nrem = (ptr + C - 1) // C

            def dflush(j, carry2, wp=wp, h=h, f0=f0):
                off = pl.multiple_of(base[h] + wp + j * C, 8)
                for f in range(3):
                    pltpu.sync_copy(
                        st.at[f0 + f, pl.ds(j * C, C)],
                        outs[f].at[h, pl.ds(off, C)])
                return carry2
            lax.fori_loop(0, nrem, dflush, 0)

        for h in range(2):
            st_h = _odd_steps(tot[h])
            npad = st_h * 2 * NS - tot[h]

            def pchunk(j, carry2, h=h, npad=npad):
                pc = pl.multiple_of((tot[h] + me + j * NW) * C, 8)
                for f in range(3):
                    pltpu.sync_copy(zc, outs[f].at[h, pl.ds(pc, C)])
                return carry2
            npad_mine = jnp.maximum((npad - me + NW - 1) // NW, 0)
            lax.fori_loop(0, npad_mine, pchunk, 0)

    return k(pk, counts)


def _propagate(fd, fs, fw, counts, cur):
    @functools.partial(
        pl.kernel,
        out_type=jax.ShapeDtypeStruct((N, D), jnp.float32),
        mesh=_mesh,
        scratch_types=[
            pltpu.VMEM_SHARED((HALF, D), jnp.float32),
            pltpu.VMEM((ZROWS, D), jnp.float32),
            pltpu.VMEM((2, 3, 2, C), jnp.int32),
            pltpu.VMEM((2, NW, 16), jnp.int32),
            pltpu.VMEM((2, 2, C, D), jnp.float32),
            pltpu.SemaphoreType.DMA,
            pltpu.SemaphoreType.DMA,
            pltpu.SemaphoreType.DMA,
            pltpu.SemaphoreType.DMA,
            pltpu.SemaphoreType.DMA,
            pltpu.SemaphoreType.DMA,
        ],
        compiler_params=_params,
    )
    def k(fd_hbm, fs_hbm, fw_hbm, counts_hbm, cur_hbm, out_hbm,
          accum, zbuf, ebuf, cbuf, rows,
          sem_e0, sem_e1, sem_g0, sem_g1, sem_s0, sem_s1):
        cid = lax.axis_index("c")
        sid = lax.axis_index("s")
        sem_e = (sem_e0, sem_e1)
        sem_g = (sem_g0, sem_g1)
        sem_s = (sem_s0, sem_s1)
        zero16 = jnp.zeros((16,), jnp.float32)

        pltpu.sync_copy(counts_hbm, cbuf)
        tot = jnp.zeros((16,), jnp.int32)
        for t in range(NW):
            tot = tot + cbuf[cid, t, pl.ds(0, 16)]
        m = _odd_steps(jnp.max(tot))

        def cbase(i):
            return jnp.minimum(sid * 2 + i * 2 * NS, CAPC - 2)

        def issue_ecopy(i, p):
            cb = cbase(i)
            pltpu.async_copy(fd_hbm.at[cid, pl.ds(cb, 2)], ebuf.at[p, 0], sem_e[p])
            pltpu.async_copy(fs_hbm.at[cid, pl.ds(cb, 2)], ebuf.at[p, 1], sem_e[p])
            pltpu.async_copy(fw_hbm.at[cid, pl.ds(cb, 2)], ebuf.at[p, 2], sem_e[p])

        def wait_ecopy(p):
            for j in range(3):
                pltpu.make_async_copy(fd_hbm.at[0, pl.ds(0, 2)],
                                      ebuf.at[p, j], sem_e[p]).wait()

        def issue_gather(p):
            for h in range(2):
                pltpu.async_copy(cur_hbm.at[ebuf.at[p, 1, h]], rows.at[p, h],
                                 sem_g[p])

        def wait_gather(p):
            for h in range(2):
                pltpu.make_async_copy(cur_hbm.at[ebuf.at[p, 1, h]],
                                      rows.at[p, h], sem_g[p]).wait()

        def issue_scatter(p):
            for h in range(2):
                pltpu.async_copy(rows.at[p, h], accum.at[ebuf.at[p, 0, h]],
                                 sem_s[p], add=True)

        def wait_scatter(p):
            for h in range(2):
                pltpu.make_async_copy(rows.at[p, h], accum.at[ebuf.at[p, 0, h]],
                                      sem_s[p]).wait()

        def process(p):
            for h in range(2):
                for v in range(C // 16):
                    wg = lax.bitcast_convert_type(
                        ebuf[p, 2, h, pl.ds(v * 16, 16)], jnp.float32)
                    for t in range(16):
                        j = v * 16 + t
                        ws = _lane_bcast(wg, t)
                        rows[p, h, j, pl.ds(0, 16)] = (
                            rows[p, h, j, pl.ds(0, 16)] * ws)
                        rows[p, h, j, pl.ds(16, 16)] = (
                            rows[p, h, j, pl.ds(16, 16)] * ws)

        def zb(i, carry):
            zbuf[i, pl.ds(0, 16)] = zero16
            zbuf[i, pl.ds(16, 16)] = zero16
            return carry
        lax.fori_loop(0, ZROWS, zb, 0)

        nblocks_mine = (NBLOCKS - sid + NS - 1) // NS

        def zero_block(i, carry):
            pltpu.sync_copy(zbuf, accum.at[pl.ds((sid + i * NS) * ZROWS, ZROWS)])
            return carry
        lax.fori_loop(0, nblocks_mine, zero_block, 0)

        issue_ecopy(0, 0)
        wait_ecopy(0)
        issue_gather(0)
        issue_ecopy(1, 1)
        plsc.subcore_barrier()

        def halfstep(i, p, scatter_wait_cond=None):
            wait_gather(1 - p)
            process(1 - p)
            issue_scatter(1 - p)
            wait_ecopy(p)
            if scatter_wait_cond is None:
                wait_scatter(p)
            else:
                @pl.when(scatter_wait_cond)
                def _():
                    wait_scatter(p)
            issue_gather(p)
            issue_ecopy(i + 1, 1 - p)

        def pair(jj, carry):
            i1 = 2 * jj + 1
            halfstep(i1, 1, scatter_wait_cond=jj > 0)
            halfstep(i1 + 1, 0)
            return carry
        lax.fori_loop(0, (m - 1) // 2, pair, 0)

        wait_gather(0)
        process(0)
        issue_scatter(0)
        wait_ecopy(1)

        @pl.when(m > 1)
        def _():
            wait_scatter(1)

        wait_scatter(0)
        plsc.subcore_barrier()

        def write_block(i, carry):
            r0 = (sid + i * NS) * ZROWS
            pltpu.sync_copy(accum.at[pl.ds(r0, ZROWS)],
                            out_hbm.at[pl.ds(cid * HALF + r0, ZROWS)])
            return carry
        lax.fori_loop(0, nblocks_mine, write_block, 0)

    return k(fd, fs, fw, counts, cur)


def _score(pid, hid, t0, t1, t2, t3):
    @functools.partial(
        pl.kernel,
        out_type=jax.ShapeDtypeStruct((B,), jnp.float32),
        mesh=_mesh,
        scratch_types=[
            pltpu.VMEM((PP,), jnp.int32),
            pltpu.VMEM((PP,), jnp.int32),
            pltpu.VMEM((PP, D), jnp.float32),
            pltpu.VMEM((PP, D), jnp.float32),
            pltpu.VMEM((PP, D), jnp.float32),
            pltpu.VMEM((PP,), jnp.float32),
            pltpu.SemaphoreType.DMA,
        ],
        compiler_params=_params,
    )
    def k(pid_hbm, hid_hbm, t0_hbm, t1_hbm, t2_hbm, t3_hbm, out_hbm,
          pidb, hidb, pacc, hacc, tmp, sb, sem):
        cid = lax.axis_index("c")
        sid = lax.axis_index("s")
        base = (cid * NS + sid) * PP
        pltpu.sync_copy(pid_hbm.at[pl.ds(base, PP)], pidb)
        pltpu.sync_copy(hid_hbm.at[pl.ds(base, PP)], hidb)

        def off(i, carry):
            hidb[pl.ds(i * 16, 16)] = hidb[pl.ds(i * 16, 16)] + NUM_P
            return carry
        lax.fori_loop(0, PP // 16, off, 0)

        def accumulate(idxb, acc):
            pltpu.async_copy(t0_hbm.at[idxb], acc, sem).wait()
            for t_hbm in (t1_hbm, t2_hbm, t3_hbm):
                pltpu.async_copy(t_hbm.at[idxb], tmp, sem).wait()

                def addv(r, carry):
                    acc[r, pl.ds(0, 16)] = acc[r, pl.ds(0, 16)] + tmp[r, pl.ds(0, 16)]
                    acc[r, pl.ds(16, 16)] = acc[r, pl.ds(16, 16)] + tmp[r, pl.ds(16, 16)]
                    return carry
                lax.fori_loop(0, PP, addv, 0)

        accumulate(pidb, pacc)
        accumulate(hidb, hacc)

        iota = lax.iota(jnp.int32, 16)

        def dotg(g, carry):
            rowidx = g * 16 + iota
            acc = jnp.zeros((16,), jnp.float32)
            for d in range(D):
                col = jnp.full((16,), d, jnp.int32)
                pc = plsc.load_gather(pacc, [rowidx, col])
                hc = plsc.load_gather(hacc, [rowidx, col])
                acc = acc + pc * hc
            sb[pl.ds(g * 16, 16)] = acc * jnp.float32(1.0 / 16.0)
            return carry
        lax.fori_loop(0, PP // 16, dotg, 0)
        pltpu.sync_copy(sb, out_hbm.at[pl.ds(base, PP)])

    return k(pid, hid, t0, t1, t2, t3)


def kernel(person_ids, hobby_ids, edge_index, edge_weight, person_emb, hobby_emb):
    dst = edge_index[0].astype(jnp.int32)
    src = edge_index[1].astype(jnp.int32)
    w = edge_weight.astype(jnp.float32)
    pk = jnp.stack([dst.reshape(NCHUNKS, C),
                    src.reshape(NCHUNKS, C),
                    lax.bitcast_convert_type(w, jnp.int32).reshape(NCHUNKS, C)],
                   axis=1)
    counts = _count(pk)
    fd, fs, fw = _compact(pk, counts)
    fd = fd.reshape(2, CAPC, C)
    fs = fs.reshape(2, CAPC, C)
    fw = fw.reshape(2, CAPC, C)
    t0 = jnp.concatenate([person_emb, hobby_emb], axis=0)
    t1 = _propagate(fd, fs, fw, counts, t0)
    t2 = _propagate(fd, fs, fw, counts, t1)
    t3 = _propagate(fd, fs, fw, counts, t2)
    return _score(person_ids.astype(jnp.int32), hobby_ids.astype(jnp.int32),
                  t0, t1, t2, t3)


# speedup vs baseline: 17.3704x; 1.0235x over previous
import functools

import jax
import jax.numpy as jnp
from jax import lax
from jax.experimental import pallas as pl
from jax.experimental.pallas import tpu as pltpu
from jax.experimental.pallas import tpu_sc as plsc

NUM_P = 60000
NUM_H = 40000
N = 100000
D = 32
E = 1600000
B = 4096
C = 128
NCHUNKS = E // C
NC = 2
NS = 16
NW = NC * NS
HALF = N // NC
ZROWS = 200
NBLOCKS = HALF // ZROWS
PP = B // NW

CNTU = (NCHUNKS + NW - 1) // NW
CAPC = 12576
CAPE = CAPC * C
STAGE = 1280
FLUSH = 1024

_mesh = plsc.VectorSubcoreMesh(core_axis_name="c", subcore_axis_name="s")
_params = pltpu.CompilerParams(use_tc_tiling_on_sc=False,
                               needs_layout_passes=False)

_GDN = lax.GatherDimensionNumbers(
    offset_dims=(), collapsed_slice_dims=(0,), start_index_map=(0,))


def _lane_bcast(vec, t):
    idx = jnp.full((16, 1), t, jnp.int32)
    return lax.gather(vec, idx, _GDN, slice_sizes=(1,),
                      mode=lax.GatherScatterMode.PROMISE_IN_BOUNDS)


def _splat(x):
    return jnp.full((16,), x, jnp.int32)


NBAT = 4
CNT4 = ((CNTU + NBAT - 1) // NBAT) | 1


def _count(dst_c):
    @functools.partial(
        pl.kernel,
        out_type=jax.ShapeDtypeStruct((2, NW, 16), jnp.int32),
        mesh=_mesh,
        scratch_types=[
            pltpu.VMEM((2, NBAT, C), jnp.int32),
            pltpu.VMEM((16,), jnp.int32),
            pltpu.SemaphoreType.DMA,
            pltpu.SemaphoreType.DMA,
        ],
        compiler_params=_params,
    )
    def k(dst_hbm, counts_hbm, ebuf, cbuf, sem_e0, sem_e1):
        cid = lax.axis_index("c")
        sid = lax.axis_index("s")
        me = cid * NS + sid
        sem_e = (sem_e0, sem_e1)
        nbatches = (NCHUNKS + NBAT - 1) // NBAT

        def issue_e(i, p):
            bb = jnp.minimum(me + i * NW, nbatches - 1)
            pltpu.async_copy(dst_hbm.at[pl.ds(bb * NBAT, NBAT)], ebuf.at[p],
                             sem_e[p])

        def wait_e(p):
            pltpu.make_async_copy(dst_hbm.at[pl.ds(0, NBAT)], ebuf.at[p],
                                  sem_e[p]).wait()

        def process(i, p, acc):
            bb = me + i * NW
            for j in range(NBAT):
                vv = _splat(((bb * NBAT + j) < NCHUNKS).astype(jnp.int32))
                for v in range(C // 16):
                    dv = ebuf[p, j, pl.ds(v * 16, 16)]
                    acc = acc + jnp.where(dv < HALF, vv, _splat(0))
            return acc

        issue_e(0, 0)
        issue_e(1, 1)
        acc0 = jnp.zeros((16,), jnp.int32)
        wait_e(0)
        acc0 = process(0, 0, acc0)
        issue_e(2, 0)

        def pair(jj, acc):
            i1 = 2 * jj + 1
            wait_e(1)
            acc = process(i1, 1, acc)
            issue_e(i1 + 2, 1)
            wait_e(0)
            acc = process(i1 + 1, 0, acc)
            issue_e(i1 + 3, 0)
            return acc
        acc0 = lax.fori_loop(0, (CNT4 - 1) // 2, pair, acc0)
        wait_e(1)
        wait_e(0)

        nlo = jnp.sum(acc0)
        nvalid = NBAT * ((nbatches - me + NW - 1) // NW)
        nhi = C * nvalid - nlo
        clo = (nlo + C - 1) // C
        chi = (nhi + C - 1) // C
        cbuf[pl.ds(0, 16)] = _splat(clo)
        pltpu.sync_copy(cbuf, counts_hbm.at[0, me])
        cbuf[pl.ds(0, 16)] = _splat(chi)
        pltpu.sync_copy(cbuf, counts_hbm.at[1, me])

    return k(dst_c)


def _odd_steps(tot):
    return ((tot + 2 * NS - 1) // (2 * NS)) | 1


def _compact(pk, counts):
    out = jax.ShapeDtypeStruct((2, CAPE), jnp.int32)

    @functools.partial(
        pl.kernel,
        out_type=(out, out, out),
        mesh=_mesh,
        scratch_types=[
            pltpu.VMEM((2, NBAT, 3, C), jnp.int32),
            pltpu.VMEM((2, NW, 16), jnp.int32),
            pltpu.VMEM((6, STAGE), jnp.int32),
            pltpu.VMEM((C,), jnp.int32),
            pltpu.SemaphoreType.DMA,
            pltpu.SemaphoreType.DMA,
        ],
        compiler_params=_params,
    )
    def k(pk_hbm, counts_hbm, fd_hbm, fs_hbm, fw_hbm,
          ebuf, cbuf, st, zc, sem_e0, sem_e1):
        cid = lax.axis_index("c")
        sid = lax.axis_index("s")
        me = cid * NS + sid
        sem_e = (sem_e0, sem_e1)
        outs = (fd_hbm, fs_hbm, fw_hbm)

        pltpu.sync_copy(counts_hbm, cbuf)
        zero16 = jnp.zeros((16,), jnp.int32)
        blo = zero16
        bhi = zero16
        tlo = zero16
        thi = zero16
        for t in range(NW):
            clv = cbuf[0, t, pl.ds(0, 16)]
            chv = cbuf[1, t, pl.ds(0, 16)]
            pred = _splat((t < me).astype(jnp.int32)) > 0
            blo = blo + jnp.where(pred, clv, zero16)
            bhi = bhi + jnp.where(pred, chv, zero16)
            tlo = tlo + clv
            thi = thi + chv
        base = (jnp.max(blo) * C, jnp.max(bhi) * C)
        tot = (jnp.max(tlo), jnp.max(thi))

        for j in range(C // 16):
            zc[pl.ds(j * 16, 16)] = zero16

        nbatches = (NCHUNKS + NBAT - 1) // NBAT

        def issue_e(i, p):
            bb = jnp.minimum(me + i * NW, nbatches - 1)
            pltpu.async_copy(pk_hbm.at[pl.ds(bb * NBAT, NBAT)], ebuf.at[p],
                             sem_e[p])

        def wait_e(p):
            pltpu.make_async_copy(pk_hbm.at[pl.ds(0, NBAT)], ebuf.at[p],
                                  sem_e[p]).wait()

        def process(i, p, carry):
            bb = me + i * NW
            for jc in range(NBAT):
                carry = process_chunk(p, jc,
                                      ((bb * NBAT + jc) < NCHUNKS), carry)
            return carry

        def process_chunk(p, jc, valid_b, carry):
            ptr_lo, ptr_hi, wp_lo, wp_hi = carry
            valid = valid_b.astype(jnp.int32)
            vmask = _splat(valid) > 0
            for v in range(C // 16):
                dv = ebuf[p, jc, 0, pl.ds(v * 16, 16)]
                sv = ebuf[p, jc, 1, pl.ds(v * 16, 16)]
                wv = ebuf[p, jc, 2, pl.ds(v * 16, 16)]
                mlo = (dv < HALF) & vmask
                mhi = (dv >= HALF) & vmask
                nlo = jnp.sum(jnp.where(mlo, _splat(1), zero16))
                nhi = valid * 16 - nlo
                plsc.store_compressed(st.at[0, pl.ds(ptr_lo, 16)], dv, mask=mlo)
                plsc.store_compressed(st.at[1, pl.ds(ptr_lo, 16)], sv, mask=mlo)
                plsc.store_compressed(st.at[2, pl.ds(ptr_lo, 16)], wv, mask=mlo)
                plsc.store_compressed(st.at[3, pl.ds(ptr_hi, 16)], dv - HALF, mask=mhi)
                plsc.store_compressed(st.at[4, pl.ds(ptr_hi, 16)], sv, mask=mhi)
                plsc.store_compressed(st.at[5, pl.ds(ptr_hi, 16)], wv, mask=mhi)
                ptr_lo = ptr_lo + nlo
                ptr_hi = ptr_hi + nhi
            for h, ptr, wp, f0 in ((0, ptr_lo, wp_lo, 0), (1, ptr_hi, wp_hi, 3)):
                do = ptr >= FLUSH

                @pl.when(do)
                def _(h=h, wp=wp, f0=f0):
                    off = pl.multiple_of(base[h] + wp, 8)
                    for f in range(3):
                        pltpu.sync_copy(st.at[f0 + f, pl.ds(0, FLUSH)],
                                        outs[f].at[h, pl.ds(off, FLUSH)])
                    for f in range(3):
                        for j in range(10):
                            st[f0 + f, pl.ds(j * 16, 16)] = (
                                st[f0 + f, pl.ds(FLUSH + j * 16, 16)])
                if h == 0:
                    ptr_lo = jnp.where(do, ptr_lo - FLUSH, ptr_lo)
                    wp_lo = jnp.where(do, wp_lo + FLUSH, wp_lo)
                else:
                    ptr_hi = jnp.where(do, ptr_hi - FLUSH, ptr_hi)
                    wp_hi = jnp.where(do, wp_hi + FLUSH, wp_hi)
            return (ptr_lo, ptr_hi, wp_lo, wp_hi)

        issue_e(0, 0)
        issue_e(1, 1)
        carry = (jnp.int32(0), jnp.int32(0), jnp.int32(0), jnp.int32(0))
        wait_e(0)
        carry = process(0, 0, carry)
        issue_e(2, 0)

        def pair(jj, carry):
            i1 = 2 * jj + 1
            wait_e(1)
            carry = process(i1, 1, carry)
            issue_e(i1 + 2, 1)
            wait_e(0)
            carry = process(i1 + 1, 0, carry)
            issue_e(i1 + 3, 0)
            return carry
        carry = lax.fori_loop(0, (CNT4 - 1) // 2, pair, carry)
        wait_e(1)
        wait_e(0)
        ptr_lo, ptr_hi, wp_lo, wp_hi = carry

        lanes = lax.iota(jnp.int32, 16)
        for ptr, wp, h, f0 in ((ptr_lo, wp_lo, 0, 0), (ptr_hi, wp_hi, 1, 3)):
            start16 = ptr & ~15
            keep = lanes < (ptr - start16)
            for f in range(3):
                vcur = st[f0 + f, pl.ds(start16, 16)]
                st[f0 + f, pl.ds(start16, 16)] = jnp.where(keep, vcur, zero16)
                for j in range(1, 8):
                    st[f0 + f, pl.ds(start16 + j * 16, 16)] = zero16
            nrem = (ptr + C - 1) // C

            def dflush(j, carry2, wp=wp, h=h, f0=f0):
                off = pl.multiple_of(base[h] + wp + j * C, 8)
                for f in range(3):
                    pltpu.sync_copy(
                        st.at[f0 + f, pl.ds(j * C, C)],
                        outs[f].at[h, pl.ds(off, C)])
                return carry2
            lax.fori_loop(0, nrem, dflush, 0)

        for h in range(2):
            st_h = _odd_steps(tot[h])
            npad = st_h * 2 * NS - tot[h]

            def pchunk(j, carry2, h=h, npad=npad):
                pc = pl.multiple_of((tot[h] + me + j * NW) * C, 8)
                for f in range(3):
                    pltpu.sync_copy(zc, outs[f].at[h, pl.ds(pc, C)])
                return carry2
            npad_mine = jnp.maximum((npad - me + NW - 1) // NW, 0)
            lax.fori_loop(0, npad_mine, pchunk, 0)

    return k(pk, counts)


def _propagate(fd, fs, fw, counts, cur):
    @functools.partial(
        pl.kernel,
        out_type=jax.ShapeDtypeStruct((N, D), jnp.float32),
        mesh=_mesh,
        scratch_types=[
            pltpu.VMEM_SHARED((HALF, D), jnp.float32),
            pltpu.VMEM((ZROWS, D), jnp.float32),
            pltpu.VMEM((2, 3, 2, C), jnp.int32),
            pltpu.VMEM((2, NW, 16), jnp.int32),
            pltpu.VMEM((2, 2, C, D), jnp.float32),
            pltpu.SemaphoreType.DMA,
            pltpu.SemaphoreType.DMA,
            pltpu.SemaphoreType.DMA,
            pltpu.SemaphoreType.DMA,
            pltpu.SemaphoreType.DMA,
            pltpu.SemaphoreType.DMA,
        ],
        compiler_params=_params,
    )
    def k(fd_hbm, fs_hbm, fw_hbm, counts_hbm, cur_hbm, out_hbm,
          accum, zbuf, ebuf, cbuf, rows,
          sem_e0, sem_e1, sem_g0, sem_g1, sem_s0, sem_s1):
        cid = lax.axis_index("c")
        sid = lax.axis_index("s")
        sem_e = (sem_e0, sem_e1)
        sem_g = (sem_g0, sem_g1)
        sem_s = (sem_s0, sem_s1)
        zero16 = jnp.zeros((16,), jnp.float32)

        pltpu.sync_copy(counts_hbm, cbuf)
        tot = jnp.zeros((16,), jnp.int32)
        for t in range(NW):
            tot = tot + cbuf[cid, t, pl.ds(0, 16)]
        m = _odd_steps(jnp.max(tot))

        def cbase(i):
            return jnp.minimum(sid * 2 + i * 2 * NS, CAPC - 2)

        def issue_ecopy(i, p):
            cb = cbase(i)
            pltpu.async_copy(fd_hbm.at[cid, pl.ds(cb, 2)], ebuf.at[p, 0], sem_e[p])
            pltpu.async_copy(fs_hbm.at[cid, pl.ds(cb, 2)], ebuf.at[p, 1], sem_e[p])
            pltpu.async_copy(fw_hbm.at[cid, pl.ds(cb, 2)], ebuf.at[p, 2], sem_e[p])

        def wait_ecopy(p):
            for j in range(3):
                pltpu.make_async_copy(fd_hbm.at[0, pl.ds(0, 2)],
                                      ebuf.at[p, j], sem_e[p]).wait()

        def issue_gather(p):
            for h in range(2):
                pltpu.async_copy(cur_hbm.at[ebuf.at[p, 1, h]], rows.at[p, h],
                                 sem_g[p])

        def wait_gather(p):
            for h in range(2):
                pltpu.make_async_copy(cur_hbm.at[ebuf.at[p, 1, h]],
                                      rows.at[p, h], sem_g[p]).wait()

        def issue_scatter(p):
            for h in range(2):
                pltpu.async_copy(rows.at[p, h], accum.at[ebuf.at[p, 0, h]],
                                 sem_s[p], add=True)

        def wait_scatter(p):
            for h in range(2):
                pltpu.make_async_copy(rows.at[p, h], accum.at[ebuf.at[p, 0, h]],
                                      sem_s[p]).wait()

        def process(p):
            for h in range(2):
                for v in range(C // 16):
                    wg = lax.bitcast_convert_type(
                        ebuf[p, 2, h, pl.ds(v * 16, 16)], jnp.float32)
                    for t in range(16):
                        j = v * 16 + t
                        ws = _lane_bcast(wg, t)
                        rows[p, h, j, pl.ds(0, 16)] = (
                            rows[p, h, j, pl.ds(0, 16)] * ws)
                        rows[p, h, j, pl.ds(16, 16)] = (
                            rows[p, h, j, pl.ds(16, 16)] * ws)

        def zb(i, carry):
            zbuf[i, pl.ds(0, 16)] = zero16
            zbuf[i, pl.ds(16, 16)] = zero16
            return carry
        lax.fori_loop(0, ZROWS, zb, 0)

        nblocks_mine = (NBLOCKS - sid + NS - 1) // NS

        def zero_block(i, carry):
            pltpu.sync_copy(zbuf, accum.at[pl.ds((sid + i * NS) * ZROWS, ZROWS)])
            return carry
        lax.fori_loop(0, nblocks_mine, zero_block, 0)

        issue_ecopy(0, 0)
        wait_ecopy(0)
        issue_gather(0)
        issue_ecopy(1, 1)
        plsc.subcore_barrier()

        def halfstep(i, p, scatter_wait_cond=None):
            wait_gather(1 - p)
            process(1 - p)
            issue_scatter(1 - p)
            wait_ecopy(p)
            if scatter_wait_cond is None:
                wait_scatter(p)
            else:
                @pl.when(scatter_wait_cond)
                def _():
                    wait_scatter(p)
            issue_gather(p)
            issue_ecopy(i + 1, 1 - p)

        def pair(jj, carry):
            i1 = 2 * jj + 1
            halfstep(i1, 1, scatter_wait_cond=jj > 0)
            halfstep(i1 + 1, 0)
            return carry
        lax.fori_loop(0, (m - 1) // 2, pair, 0)

        wait_gather(0)
        process(0)
        issue_scatter(0)
        wait_ecopy(1)

        @pl.when(m > 1)
        def _():
            wait_scatter(1)

        wait_scatter(0)
        plsc.subcore_barrier()

        def write_block(i, carry):
            r0 = (sid + i * NS) * ZROWS
            pltpu.sync_copy(accum.at[pl.ds(r0, ZROWS)],
                            out_hbm.at[pl.ds(cid * HALF + r0, ZROWS)])
            return carry
        lax.fori_loop(0, nblocks_mine, write_block, 0)

    return k(fd, fs, fw, counts, cur)


def _score(pid, hid, t0, t1, t2, t3):
    @functools.partial(
        pl.kernel,
        out_type=jax.ShapeDtypeStruct((B,), jnp.float32),
        mesh=_mesh,
        scratch_types=[
            pltpu.VMEM((PP,), jnp.int32),
            pltpu.VMEM((PP,), jnp.int32),
            pltpu.VMEM((PP, D), jnp.float32),
            pltpu.VMEM((PP, D), jnp.float32),
            pltpu.VMEM((PP, D), jnp.float32),
            pltpu.VMEM((PP,), jnp.float32),
            pltpu.SemaphoreType.DMA,
        ],
        compiler_params=_params,
    )
    def k(pid_hbm, hid_hbm, t0_hbm, t1_hbm, t2_hbm, t3_hbm, out_hbm,
          pidb, hidb, pacc, hacc, tmp, sb, sem):
        cid = lax.axis_index("c")
        sid = lax.axis_index("s")
        base = (cid * NS + sid) * PP
        pltpu.sync_copy(pid_hbm.at[pl.ds(base, PP)], pidb)
        pltpu.sync_copy(hid_hbm.at[pl.ds(base, PP)], hidb)

        def off(i, carry):
            hidb[pl.ds(i * 16, 16)] = hidb[pl.ds(i * 16, 16)] + NUM_P
            return carry
        lax.fori_loop(0, PP // 16, off, 0)

        def accumulate(idxb, acc):
            pltpu.async_copy(t0_hbm.at[idxb], acc, sem).wait()
            for t_hbm in (t1_hbm, t2_hbm, t3_hbm):
                pltpu.async_copy(t_hbm.at[idxb], tmp, sem).wait()

                def addv(r, carry):
                    acc[r, pl.ds(0, 16)] = acc[r, pl.ds(0, 16)] + tmp[r, pl.ds(0, 16)]
                    acc[r, pl.ds(16, 16)] = acc[r, pl.ds(16, 16)] + tmp[r, pl.ds(16, 16)]
                    return carry
                lax.fori_loop(0, PP, addv, 0)

        accumulate(pidb, pacc)
        accumulate(hidb, hacc)

        iota = lax.iota(jnp.int32, 16)

        def dotg(g, carry):
            rowidx = g * 16 + iota
            acc = jnp.zeros((16,), jnp.float32)
            for d in range(D):
                col = jnp.full((16,), d, jnp.int32)
                pc = plsc.load_gather(pacc, [rowidx, col])
                hc = plsc.load_gather(hacc, [rowidx, col])
                acc = acc + pc * hc
            sb[pl.ds(g * 16, 16)] = acc * jnp.float32(1.0 / 16.0)
            return carry
        lax.fori_loop(0, PP // 16, dotg, 0)
        pltpu.sync_copy(sb, out_hbm.at[pl.ds(base, PP)])

    return k(pid, hid, t0, t1, t2, t3)


def kernel(person_ids, hobby_ids, edge_index, edge_weight, person_emb, hobby_emb):
    dst = edge_index[0].astype(jnp.int32)
    src = edge_index[1].astype(jnp.int32)
    w = edge_weight.astype(jnp.float32)
    pk = jnp.stack([dst.reshape(NCHUNKS, C),
                    src.reshape(NCHUNKS, C),
                    lax.bitcast_convert_type(w, jnp.int32).reshape(NCHUNKS, C)],
                   axis=1)
    counts = _count(dst.reshape(NCHUNKS, C))
    fd, fs, fw = _compact(pk, counts)
    fd = fd.reshape(2, CAPC, C)
    fs = fs.reshape(2, CAPC, C)
    fw = fw.reshape(2, CAPC, C)
    t0 = jnp.concatenate([person_emb, hobby_emb], axis=0)
    t1 = _propagate(fd, fs, fw, counts, t0)
    t2 = _propagate(fd, fs, fw, counts, t1)
    t3 = _propagate(fd, fs, fw, counts, t2)
    return _score(person_ids.astype(jnp.int32), hobby_ids.astype(jnp.int32),
                  t0, t1, t2, t3)


# speedup vs baseline: 18.2302x; 1.0495x over previous
import functools

import jax
import jax.numpy as jnp
from jax import lax
from jax.experimental import pallas as pl
from jax.experimental.pallas import tpu as pltpu
from jax.experimental.pallas import tpu_sc as plsc

NUM_P = 60000
NUM_H = 40000
N = 100000
D = 32
E = 1600000
B = 4096
C = 128
NCHUNKS = E // C
NC = 2
NS = 16
NW = NC * NS
HALF = N // NC
ZROWS = 200
NBLOCKS = HALF // ZROWS
PP = B // NW

CNTU = (NCHUNKS + NW - 1) // NW
CAPC = 12576
CAPE = CAPC * C
STAGE = 1280
FLUSH = 1024

_mesh = plsc.VectorSubcoreMesh(core_axis_name="c", subcore_axis_name="s")
_params = pltpu.CompilerParams(use_tc_tiling_on_sc=False,
                               needs_layout_passes=False)

_GDN = lax.GatherDimensionNumbers(
    offset_dims=(), collapsed_slice_dims=(0,), start_index_map=(0,))


def _lane_bcast(vec, t):
    idx = jnp.full((16, 1), t, jnp.int32)
    return lax.gather(vec, idx, _GDN, slice_sizes=(1,),
                      mode=lax.GatherScatterMode.PROMISE_IN_BOUNDS)


def _splat(x):
    return jnp.full((16,), x, jnp.int32)


NBAT = 4
CNT4 = ((CNTU + NBAT - 1) // NBAT) | 1


def _count(dst_c):
    @functools.partial(
        pl.kernel,
        out_type=jax.ShapeDtypeStruct((2, NW, 16), jnp.int32),
        mesh=_mesh,
        scratch_types=[
            pltpu.VMEM((2, NBAT, C), jnp.int32),
            pltpu.VMEM((16,), jnp.int32),
            pltpu.SemaphoreType.DMA,
            pltpu.SemaphoreType.DMA,
        ],
        compiler_params=_params,
    )
    def k(dst_hbm, counts_hbm, ebuf, cbuf, sem_e0, sem_e1):
        cid = lax.axis_index("c")
        sid = lax.axis_index("s")
        me = cid * NS + sid
        sem_e = (sem_e0, sem_e1)
        nbatches = (NCHUNKS + NBAT - 1) // NBAT

        def issue_e(i, p):
            bb = jnp.minimum(me + i * NW, nbatches - 1)
            pltpu.async_copy(dst_hbm.at[pl.ds(bb * NBAT, NBAT)], ebuf.at[p],
                             sem_e[p])

        def wait_e(p):
            pltpu.make_async_copy(dst_hbm.at[pl.ds(0, NBAT)], ebuf.at[p],
                                  sem_e[p]).wait()

        def process(i, p, acc):
            bb = me + i * NW
            for j in range(NBAT):
                vv = _splat(((bb * NBAT + j) < NCHUNKS).astype(jnp.int32))
                for v in range(C // 16):
                    dv = ebuf[p, j, pl.ds(v * 16, 16)]
                    acc = acc + jnp.where(dv < HALF, vv, _splat(0))
            return acc

        issue_e(0, 0)
        issue_e(1, 1)
        acc0 = jnp.zeros((16,), jnp.int32)
        wait_e(0)
        acc0 = process(0, 0, acc0)
        issue_e(2, 0)

        def pair(jj, acc):
            i1 = 2 * jj + 1
            wait_e(1)
            acc = process(i1, 1, acc)
            issue_e(i1 + 2, 1)
            wait_e(0)
            acc = process(i1 + 1, 0, acc)
            issue_e(i1 + 3, 0)
            return acc
        acc0 = lax.fori_loop(0, (CNT4 - 1) // 2, pair, acc0)
        wait_e(1)
        wait_e(0)

        nlo = jnp.sum(acc0)
        nvalid = NBAT * ((nbatches - me + NW - 1) // NW)
        nhi = C * nvalid - nlo
        clo = (nlo + C - 1) // C
        chi = (nhi + C - 1) // C
        cbuf[pl.ds(0, 16)] = _splat(clo)
        pltpu.sync_copy(cbuf, counts_hbm.at[0, me])
        cbuf[pl.ds(0, 16)] = _splat(chi)
        pltpu.sync_copy(cbuf, counts_hbm.at[1, me])

    return k(dst_c)


def _odd_steps(tot):
    return ((tot + 2 * NS - 1) // (2 * NS)) | 1


def _compact(pk, counts):
    out = jax.ShapeDtypeStruct((2, CAPE), jnp.int32)

    @functools.partial(
        pl.kernel,
        out_type=(out, out, out),
        mesh=_mesh,
        scratch_types=[
            pltpu.VMEM((2, NBAT, 3, C), jnp.int32),
            pltpu.VMEM((2, NW, 16), jnp.int32),
            pltpu.VMEM((6, STAGE), jnp.int32),
            pltpu.VMEM((C,), jnp.int32),
            pltpu.SemaphoreType.DMA,
            pltpu.SemaphoreType.DMA,
        ],
        compiler_params=_params,
    )
    def k(pk_hbm, counts_hbm, fd_hbm, fs_hbm, fw_hbm,
          ebuf, cbuf, st, zc, sem_e0, sem_e1):
        cid = lax.axis_index("c")
        sid = lax.axis_index("s")
        me = cid * NS + sid
        sem_e = (sem_e0, sem_e1)
        outs = (fd_hbm, fs_hbm, fw_hbm)

        pltpu.sync_copy(counts_hbm, cbuf)
        zero16 = jnp.zeros((16,), jnp.int32)
        blo = zero16
        bhi = zero16
        tlo = zero16
        thi = zero16
        for t in range(NW):
            clv = cbuf[0, t, pl.ds(0, 16)]
            chv = cbuf[1, t, pl.ds(0, 16)]
            pred = _splat((t < me).astype(jnp.int32)) > 0
            blo = blo + jnp.where(pred, clv, zero16)
            bhi = bhi + jnp.where(pred, chv, zero16)
            tlo = tlo + clv
            thi = thi + chv
        base = (jnp.max(blo) * C, jnp.max(bhi) * C)
        tot = (jnp.max(tlo), jnp.max(thi))

        for j in range(C // 16):
            zc[pl.ds(j * 16, 16)] = zero16

        nbatches = (NCHUNKS + NBAT - 1) // NBAT

        def issue_e(i, p):
            bb = jnp.minimum(me + i * NW, nbatches - 1)
            pltpu.async_copy(pk_hbm.at[pl.ds(bb * NBAT, NBAT)], ebuf.at[p],
                             sem_e[p])

        def wait_e(p):
            pltpu.make_async_copy(pk_hbm.at[pl.ds(0, NBAT)], ebuf.at[p],
                                  sem_e[p]).wait()

        def process(i, p, carry):
            bb = me + i * NW
            for jc in range(NBAT):
                carry = process_chunk(p, jc,
                                      ((bb * NBAT + jc) < NCHUNKS), carry)
            return carry

        def process_chunk(p, jc, valid_b, carry):
            ptr_lo, ptr_hi, wp_lo, wp_hi = carry
            valid = valid_b.astype(jnp.int32)
            vmask = _splat(valid) > 0
            for v in range(C // 16):
                dv = ebuf[p, jc, 0, pl.ds(v * 16, 16)]
                sv = ebuf[p, jc, 1, pl.ds(v * 16, 16)]
                wv = ebuf[p, jc, 2, pl.ds(v * 16, 16)]
                mlo = (dv < HALF) & vmask
                mhi = (dv >= HALF) & vmask
                nlo = jnp.sum(jnp.where(mlo, _splat(1), zero16))
                nhi = valid * 16 - nlo
                plsc.store_compressed(st.at[0, pl.ds(ptr_lo, 16)], dv, mask=mlo)
                plsc.store_compressed(st.at[1, pl.ds(ptr_lo, 16)], sv, mask=mlo)
                plsc.store_compressed(st.at[2, pl.ds(ptr_lo, 16)], wv, mask=mlo)
                plsc.store_compressed(st.at[3, pl.ds(ptr_hi, 16)], dv - HALF, mask=mhi)
                plsc.store_compressed(st.at[4, pl.ds(ptr_hi, 16)], sv, mask=mhi)
                plsc.store_compressed(st.at[5, pl.ds(ptr_hi, 16)], wv, mask=mhi)
                ptr_lo = ptr_lo + nlo
                ptr_hi = ptr_hi + nhi
            for h, ptr, wp, f0 in ((0, ptr_lo, wp_lo, 0), (1, ptr_hi, wp_hi, 3)):
                do = ptr >= FLUSH

                @pl.when(do)
                def _(h=h, wp=wp, f0=f0):
                    off = pl.multiple_of(base[h] + wp, 8)
                    for f in range(3):
                        pltpu.sync_copy(st.at[f0 + f, pl.ds(0, FLUSH)],
                                        outs[f].at[h, pl.ds(off, FLUSH)])
                    for f in range(3):
                        for j in range(10):
                            st[f0 + f, pl.ds(j * 16, 16)] = (
                                st[f0 + f, pl.ds(FLUSH + j * 16, 16)])
                if h == 0:
                    ptr_lo = jnp.where(do, ptr_lo - FLUSH, ptr_lo)
                    wp_lo = jnp.where(do, wp_lo + FLUSH, wp_lo)
                else:
                    ptr_hi = jnp.where(do, ptr_hi - FLUSH, ptr_hi)
                    wp_hi = jnp.where(do, wp_hi + FLUSH, wp_hi)
            return (ptr_lo, ptr_hi, wp_lo, wp_hi)

        issue_e(0, 0)
        issue_e(1, 1)
        carry = (jnp.int32(0), jnp.int32(0), jnp.int32(0), jnp.int32(0))
        wait_e(0)
        carry = process(0, 0, carry)
        issue_e(2, 0)

        def pair(jj, carry):
            i1 = 2 * jj + 1
            wait_e(1)
            carry = process(i1, 1, carry)
            issue_e(i1 + 2, 1)
            wait_e(0)
            carry = process(i1 + 1, 0, carry)
            issue_e(i1 + 3, 0)
            return carry
        carry = lax.fori_loop(0, (CNT4 - 1) // 2, pair, carry)
        wait_e(1)
        wait_e(0)
        ptr_lo, ptr_hi, wp_lo, wp_hi = carry

        lanes = lax.iota(jnp.int32, 16)
        for ptr, wp, h, f0 in ((ptr_lo, wp_lo, 0, 0), (ptr_hi, wp_hi, 1, 3)):
            start16 = ptr & ~15
            keep = lanes < (ptr - start16)
            for f in range(3):
                vcur = st[f0 + f, pl.ds(start16, 16)]
                st[f0 + f, pl.ds(start16, 16)] = jnp.where(keep, vcur, zero16)
                for j in range(1, 8):
                    st[f0 + f, pl.ds(start16 + j * 16, 16)] = zero16
            nrem = (ptr + C - 1) // C

            def dflush(j, carry2, wp=wp, h=h, f0=f0):
                off = pl.multiple_of(base[h] + wp + j * C, 8)
                for f in range(3):
                    pltpu.sync_copy(
                        st.at[f0 + f, pl.ds(j * C, C)],
                        outs[f].at[h, pl.ds(off, C)])
                return carry2
            lax.fori_loop(0, nrem, dflush, 0)

        for h in range(2):
            st_h = _odd_steps(tot[h])
            npad = st_h * 2 * NS - tot[h]

            def pchunk(j, carry2, h=h, npad=npad):
                pc = pl.multiple_of((tot[h] + me + j * NW) * C, 8)
                for f in range(3):
                    pltpu.sync_copy(zc, outs[f].at[h, pl.ds(pc, C)])
                return carry2
            npad_mine = jnp.maximum((npad - me + NW - 1) // NW, 0)
            lax.fori_loop(0, npad_mine, pchunk, 0)

    return k(pk, counts)


def _propagate(fd, fs, fw, counts, cur):
    @functools.partial(
        pl.kernel,
        out_type=jax.ShapeDtypeStruct((N, D), jnp.float32),
        mesh=_mesh,
        scratch_types=[
            pltpu.VMEM_SHARED((HALF, D), jnp.float32),
            pltpu.VMEM((ZROWS, D), jnp.float32),
            pltpu.VMEM((2, 3, 2, C), jnp.int32),
            pltpu.VMEM((2, NW, 16), jnp.int32),
            pltpu.VMEM((2, 2, C, D), jnp.float32),
            pltpu.SemaphoreType.DMA,
            pltpu.SemaphoreType.DMA,
            pltpu.SemaphoreType.DMA,
            pltpu.SemaphoreType.DMA,
            pltpu.SemaphoreType.DMA,
            pltpu.SemaphoreType.DMA,
        ],
        compiler_params=_params,
    )
    def k(fd_hbm, fs_hbm, fw_hbm, counts_hbm, cur_hbm, out_hbm,
          accum, zbuf, ebuf, cbuf, rows,
          sem_e0, sem_e1, sem_g0, sem_g1, sem_s0, sem_s1):
        cid = lax.axis_index("c")
        sid = lax.axis_index("s")
        sem_e = (sem_e0, sem_e1)
        sem_g = (sem_g0, sem_g1)
        sem_s = (sem_s0, sem_s1)
        zero16 = jnp.zeros((16,), jnp.float32)

        pltpu.sync_copy(counts_hbm, cbuf)
        tot = jnp.zeros((16,), jnp.int32)
        for t in range(NW):
            tot = tot + cbuf[cid, t, pl.ds(0, 16)]
        m = _odd_steps(jnp.max(tot))

        def cbase(i):
            return jnp.minimum(sid * 2 + i * 2 * NS, CAPC - 2)

        def issue_ecopy(i, p):
            cb = cbase(i)
            pltpu.async_copy(fd_hbm.at[cid, pl.ds(cb, 2)], ebuf.at[p, 0], sem_e[p])
            pltpu.async_copy(fs_hbm.at[cid, pl.ds(cb, 2)], ebuf.at[p, 1], sem_e[p])
            pltpu.async_copy(fw_hbm.at[cid, pl.ds(cb, 2)], ebuf.at[p, 2], sem_e[p])

        def wait_ecopy(p):
            for j in range(3):
                pltpu.make_async_copy(fd_hbm.at[0, pl.ds(0, 2)],
                                      ebuf.at[p, j], sem_e[p]).wait()

        def issue_gather(p):
            for h in range(2):
                pltpu.async_copy(cur_hbm.at[ebuf.at[p, 1, h]], rows.at[p, h],
                                 sem_g[p])

        def wait_gather_h(p, h):
            pltpu.make_async_copy(cur_hbm.at[ebuf.at[p, 1, h]],
                                  rows.at[p, h], sem_g[p]).wait()

        def issue_scatter_h(p, h):
            pltpu.async_copy(rows.at[p, h], accum.at[ebuf.at[p, 0, h]],
                             sem_s[p], add=True)

        def wait_gather(p):
            for h in range(2):
                wait_gather_h(p, h)

        def issue_scatter(p):
            for h in range(2):
                issue_scatter_h(p, h)

        def wait_scatter(p):
            for h in range(2):
                pltpu.make_async_copy(rows.at[p, h], accum.at[ebuf.at[p, 0, h]],
                                      sem_s[p]).wait()

        def process_h(p, h):
            for v in range(C // 16):
                wg = lax.bitcast_convert_type(
                    ebuf[p, 2, h, pl.ds(v * 16, 16)], jnp.float32)
                for t in range(16):
                    j = v * 16 + t
                    ws = _lane_bcast(wg, t)
                    rows[p, h, j, pl.ds(0, 16)] = (
                        rows[p, h, j, pl.ds(0, 16)] * ws)
                    rows[p, h, j, pl.ds(16, 16)] = (
                        rows[p, h, j, pl.ds(16, 16)] * ws)

        def process(p):
            for h in range(2):
                process_h(p, h)

        def zb(i, carry):
            zbuf[i, pl.ds(0, 16)] = zero16
            zbuf[i, pl.ds(16, 16)] = zero16
            return carry
        lax.fori_loop(0, ZROWS, zb, 0)

        nblocks_mine = (NBLOCKS - sid + NS - 1) // NS

        def zero_block(i, carry):
            pltpu.sync_copy(zbuf, accum.at[pl.ds((sid + i * NS) * ZROWS, ZROWS)])
            return carry
        lax.fori_loop(0, nblocks_mine, zero_block, 0)

        issue_ecopy(0, 0)
        wait_ecopy(0)
        issue_gather(0)
        issue_ecopy(1, 1)
        plsc.subcore_barrier()

        def halfstep(i, p, scatter_wait_cond=None):
            for h in range(2):
                wait_gather_h(1 - p, h)
                process_h(1 - p, h)
                issue_scatter_h(1 - p, h)
            wait_ecopy(p)
            if scatter_wait_cond is None:
                wait_scatter(p)
            else:
                @pl.when(scatter_wait_cond)
                def _():
                    wait_scatter(p)
            issue_gather(p)
            issue_ecopy(i + 1, 1 - p)

        def pair(jj, carry):
            i1 = 2 * jj + 1
            halfstep(i1, 1, scatter_wait_cond=jj > 0)
            halfstep(i1 + 1, 0)
            return carry
        lax.fori_loop(0, (m - 1) // 2, pair, 0)

        wait_gather(0)
        process(0)
        issue_scatter(0)
        wait_ecopy(1)

        @pl.when(m > 1)
        def _():
            wait_scatter(1)

        wait_scatter(0)
        plsc.subcore_barrier()

        def write_block(i, carry):
            r0 = (sid + i * NS) * ZROWS
            pltpu.sync_copy(accum.at[pl.ds(r0, ZROWS)],
                            out_hbm.at[pl.ds(cid * HALF + r0, ZROWS)])
            return carry
        lax.fori_loop(0, nblocks_mine, write_block, 0)

    return k(fd, fs, fw, counts, cur)


def _score(pid, hid, t0, t1, t2, t3):
    @functools.partial(
        pl.kernel,
        out_type=jax.ShapeDtypeStruct((B,), jnp.float32),
        mesh=_mesh,
        scratch_types=[
            pltpu.VMEM((PP,), jnp.int32),
            pltpu.VMEM((PP,), jnp.int32),
            pltpu.VMEM((PP, D), jnp.float32),
            pltpu.VMEM((PP, D), jnp.float32),
            pltpu.VMEM((PP, D), jnp.float32),
            pltpu.VMEM((PP,), jnp.float32),
            pltpu.SemaphoreType.DMA,
        ],
        compiler_params=_params,
    )
    def k(pid_hbm, hid_hbm, t0_hbm, t1_hbm, t2_hbm, t3_hbm, out_hbm,
          pidb, hidb, pacc, hacc, tmp, sb, sem):
        cid = lax.axis_index("c")
        sid = lax.axis_index("s")
        base = (cid * NS + sid) * PP
        pltpu.sync_copy(pid_hbm.at[pl.ds(base, PP)], pidb)
        pltpu.sync_copy(hid_hbm.at[pl.ds(base, PP)], hidb)

        def off(i, carry):
            hidb[pl.ds(i * 16, 16)] = hidb[pl.ds(i * 16, 16)] + NUM_P
            return carry
        lax.fori_loop(0, PP // 16, off, 0)

        def accumulate(idxb, acc):
            pltpu.async_copy(t0_hbm.at[idxb], acc, sem).wait()
            for t_hbm in (t1_hbm, t2_hbm, t3_hbm):
                pltpu.async_copy(t_hbm.at[idxb], tmp, sem).wait()

                def addv(r, carry):
                    acc[r, pl.ds(0, 16)] = acc[r, pl.ds(0, 16)] + tmp[r, pl.ds(0, 16)]
                    acc[r, pl.ds(16, 16)] = acc[r, pl.ds(16, 16)] + tmp[r, pl.ds(16, 16)]
                    return carry
                lax.fori_loop(0, PP, addv, 0)

        accumulate(pidb, pacc)
        accumulate(hidb, hacc)

        iota = lax.iota(jnp.int32, 16)

        def dotg(g, carry):
            rowidx = g * 16 + iota
            acc = jnp.zeros((16,), jnp.float32)
            for d in range(D):
                col = jnp.full((16,), d, jnp.int32)
                pc = plsc.load_gather(pacc, [rowidx, col])
                hc = plsc.load_gather(hacc, [rowidx, col])
                acc = acc + pc * hc
            sb[pl.ds(g * 16, 16)] = acc * jnp.float32(1.0 / 16.0)
            return carry
        lax.fori_loop(0, PP // 16, dotg, 0)
        pltpu.sync_copy(sb, out_hbm.at[pl.ds(base, PP)])

    return k(pid, hid, t0, t1, t2, t3)


def kernel(person_ids, hobby_ids, edge_index, edge_weight, person_emb, hobby_emb):
    dst = edge_index[0].astype(jnp.int32)
    src = edge_index[1].astype(jnp.int32)
    w = edge_weight.astype(jnp.float32)
    pk = jnp.stack([dst.reshape(NCHUNKS, C),
                    src.reshape(NCHUNKS, C),
                    lax.bitcast_convert_type(w, jnp.int32).reshape(NCHUNKS, C)],
                   axis=1)
    counts = _count(dst.reshape(NCHUNKS, C))
    fd, fs, fw = _compact(pk, counts)
    fd = fd.reshape(2, CAPC, C)
    fs = fs.reshape(2, CAPC, C)
    fw = fw.reshape(2, CAPC, C)
    t0 = jnp.concatenate([person_emb, hobby_emb], axis=0)
    t1 = _propagate(fd, fs, fw, counts, t0)
    t2 = _propagate(fd, fs, fw, counts, t1)
    t3 = _propagate(fd, fs, fw, counts, t2)
    return _score(person_ids.astype(jnp.int32), hobby_ids.astype(jnp.int32),
                  t0, t1, t2, t3)


# speedup vs baseline: 20.0599x; 1.1004x over previous
import functools

import jax
import jax.numpy as jnp
from jax import lax
from jax.experimental import pallas as pl
from jax.experimental.pallas import tpu as pltpu
from jax.experimental.pallas import tpu_sc as plsc

NUM_P = 60000
NUM_H = 40000
N = 100000
D = 32
E = 1600000
B = 4096
C = 128
NCHUNKS = E // C
NC = 2
NS = 16
NW = NC * NS
HALF = N // NC
ZROWS = 200
NBLOCKS = HALF // ZROWS
PP = B // NW

CNTU = (NCHUNKS + NW - 1) // NW
CAPC = 12576
CAPE = CAPC * C
STAGE = 1280
FLUSH = 1024

_mesh = plsc.VectorSubcoreMesh(core_axis_name="c", subcore_axis_name="s")
_params = pltpu.CompilerParams(use_tc_tiling_on_sc=False,
                               needs_layout_passes=False)

_GDN = lax.GatherDimensionNumbers(
    offset_dims=(), collapsed_slice_dims=(0,), start_index_map=(0,))


def _lane_bcast(vec, t):
    idx = jnp.full((16, 1), t, jnp.int32)
    return lax.gather(vec, idx, _GDN, slice_sizes=(1,),
                      mode=lax.GatherScatterMode.PROMISE_IN_BOUNDS)


def _splat(x):
    return jnp.full((16,), x, jnp.int32)


NBAT = 4
CNT4 = ((CNTU + NBAT - 1) // NBAT) | 1


def _count(dst_c):
    @functools.partial(
        pl.kernel,
        out_type=jax.ShapeDtypeStruct((2, NW, 16), jnp.int32),
        mesh=_mesh,
        scratch_types=[
            pltpu.VMEM((2, NBAT, C), jnp.int32),
            pltpu.VMEM((16,), jnp.int32),
            pltpu.SemaphoreType.DMA,
            pltpu.SemaphoreType.DMA,
        ],
        compiler_params=_params,
    )
    def k(dst_hbm, counts_hbm, ebuf, cbuf, sem_e0, sem_e1):
        cid = lax.axis_index("c")
        sid = lax.axis_index("s")
        me = cid * NS + sid
        sem_e = (sem_e0, sem_e1)
        nbatches = (NCHUNKS + NBAT - 1) // NBAT

        def issue_e(i, p):
            bb = jnp.minimum(me + i * NW, nbatches - 1)
            pltpu.async_copy(dst_hbm.at[pl.ds(bb * NBAT, NBAT)], ebuf.at[p],
                             sem_e[p])

        def wait_e(p):
            pltpu.make_async_copy(dst_hbm.at[pl.ds(0, NBAT)], ebuf.at[p],
                                  sem_e[p]).wait()

        def process(i, p, acc):
            bb = me + i * NW
            for j in range(NBAT):
                vv = _splat(((bb * NBAT + j) < NCHUNKS).astype(jnp.int32))
                for v in range(C // 16):
                    dv = ebuf[p, j, pl.ds(v * 16, 16)]
                    acc = acc + jnp.where(dv < HALF, vv, _splat(0))
            return acc

        issue_e(0, 0)
        issue_e(1, 1)
        acc0 = jnp.zeros((16,), jnp.int32)
        wait_e(0)
        acc0 = process(0, 0, acc0)
        issue_e(2, 0)

        def pair(jj, acc):
            i1 = 2 * jj + 1
            wait_e(1)
            acc = process(i1, 1, acc)
            issue_e(i1 + 2, 1)
            wait_e(0)
            acc = process(i1 + 1, 0, acc)
            issue_e(i1 + 3, 0)
            return acc
        acc0 = lax.fori_loop(0, (CNT4 - 1) // 2, pair, acc0)
        wait_e(1)
        wait_e(0)

        nlo = jnp.sum(acc0)
        nvalid = NBAT * ((nbatches - me + NW - 1) // NW)
        nhi = C * nvalid - nlo
        clo = (nlo + C - 1) // C
        chi = (nhi + C - 1) // C
        cbuf[pl.ds(0, 16)] = _splat(clo)
        pltpu.sync_copy(cbuf, counts_hbm.at[0, me])
        cbuf[pl.ds(0, 16)] = _splat(chi)
        pltpu.sync_copy(cbuf, counts_hbm.at[1, me])

    return k(dst_c)


def _odd_steps(tot):
    return ((tot + 2 * NS - 1) // (2 * NS)) | 1


def _compact(pk, counts):
    out = jax.ShapeDtypeStruct((2, CAPE), jnp.int32)

    @functools.partial(
        pl.kernel,
        out_type=(out, out, out),
        mesh=_mesh,
        scratch_types=[
            pltpu.VMEM((2, NBAT, 3, C), jnp.int32),
            pltpu.VMEM((2, NW, 16), jnp.int32),
            pltpu.VMEM((6, STAGE), jnp.int32),
            pltpu.VMEM((C,), jnp.int32),
            pltpu.SemaphoreType.DMA,
            pltpu.SemaphoreType.DMA,
        ],
        compiler_params=_params,
    )
    def k(pk_hbm, counts_hbm, fd_hbm, fs_hbm, fw_hbm,
          ebuf, cbuf, st, zc, sem_e0, sem_e1):
        cid = lax.axis_index("c")
        sid = lax.axis_index("s")
        me = cid * NS + sid
        sem_e = (sem_e0, sem_e1)
        outs = (fd_hbm, fs_hbm, fw_hbm)

        pltpu.sync_copy(counts_hbm, cbuf)
        zero16 = jnp.zeros((16,), jnp.int32)
        blo = zero16
        bhi = zero16
        tlo = zero16
        thi = zero16
        for t in range(NW):
            clv = cbuf[0, t, pl.ds(0, 16)]
            chv = cbuf[1, t, pl.ds(0, 16)]
            pred = _splat((t < me).astype(jnp.int32)) > 0
            blo = blo + jnp.where(pred, clv, zero16)
            bhi = bhi + jnp.where(pred, chv, zero16)
            tlo = tlo + clv
            thi = thi + chv
        base = (jnp.max(blo) * C, jnp.max(bhi) * C)
        tot = (jnp.max(tlo), jnp.max(thi))

        for j in range(C // 16):
            zc[pl.ds(j * 16, 16)] = zero16

        nbatches = (NCHUNKS + NBAT - 1) // NBAT

        def issue_e(i, p):
            bb = jnp.minimum(me + i * NW, nbatches - 1)
            pltpu.async_copy(pk_hbm.at[pl.ds(bb * NBAT, NBAT)], ebuf.at[p],
                             sem_e[p])

        def wait_e(p):
            pltpu.make_async_copy(pk_hbm.at[pl.ds(0, NBAT)], ebuf.at[p],
                                  sem_e[p]).wait()

        def process(i, p, carry):
            bb = me + i * NW
            for jc in range(NBAT):
                carry = process_chunk(p, jc,
                                      ((bb * NBAT + jc) < NCHUNKS), carry)
            return carry

        def process_chunk(p, jc, valid_b, carry):
            ptr_lo, ptr_hi, wp_lo, wp_hi = carry
            valid = valid_b.astype(jnp.int32)
            vmask = _splat(valid) > 0
            for v in range(C // 16):
                dv = ebuf[p, jc, 0, pl.ds(v * 16, 16)]
                sv = ebuf[p, jc, 1, pl.ds(v * 16, 16)]
                wv = ebuf[p, jc, 2, pl.ds(v * 16, 16)]
                mlo = (dv < HALF) & vmask
                mhi = (dv >= HALF) & vmask
                nlo = jnp.sum(jnp.where(mlo, _splat(1), zero16))
                nhi = valid * 16 - nlo
                plsc.store_compressed(st.at[0, pl.ds(ptr_lo, 16)], dv, mask=mlo)
                plsc.store_compressed(st.at[1, pl.ds(ptr_lo, 16)], sv, mask=mlo)
                plsc.store_compressed(st.at[2, pl.ds(ptr_lo, 16)], wv, mask=mlo)
                plsc.store_compressed(st.at[3, pl.ds(ptr_hi, 16)], dv - HALF, mask=mhi)
                plsc.store_compressed(st.at[4, pl.ds(ptr_hi, 16)], sv, mask=mhi)
                plsc.store_compressed(st.at[5, pl.ds(ptr_hi, 16)], wv, mask=mhi)
                ptr_lo = ptr_lo + nlo
                ptr_hi = ptr_hi + nhi
            for h, ptr, wp, f0 in ((0, ptr_lo, wp_lo, 0), (1, ptr_hi, wp_hi, 3)):
                do = ptr >= FLUSH

                @pl.when(do)
                def _(h=h, wp=wp, f0=f0):
                    off = pl.multiple_of(base[h] + wp, 8)
                    for f in range(3):
                        pltpu.sync_copy(st.at[f0 + f, pl.ds(0, FLUSH)],
                                        outs[f].at[h, pl.ds(off, FLUSH)])
                    for f in range(3):
                        for j in range(10):
                            st[f0 + f, pl.ds(j * 16, 16)] = (
                                st[f0 + f, pl.ds(FLUSH + j * 16, 16)])
                if h == 0:
                    ptr_lo = jnp.where(do, ptr_lo - FLUSH, ptr_lo)
                    wp_lo = jnp.where(do, wp_lo + FLUSH, wp_lo)
                else:
                    ptr_hi = jnp.where(do, ptr_hi - FLUSH, ptr_hi)
                    wp_hi = jnp.where(do, wp_hi + FLUSH, wp_hi)
            return (ptr_lo, ptr_hi, wp_lo, wp_hi)

        issue_e(0, 0)
        issue_e(1, 1)
        carry = (jnp.int32(0), jnp.int32(0), jnp.int32(0), jnp.int32(0))
        wait_e(0)
        carry = process(0, 0, carry)
        issue_e(2, 0)

        def pair(jj, carry):
            i1 = 2 * jj + 1
            wait_e(1)
            carry = process(i1, 1, carry)
            issue_e(i1 + 2, 1)
            wait_e(0)
            carry = process(i1 + 1, 0, carry)
            issue_e(i1 + 3, 0)
            return carry
        carry = lax.fori_loop(0, (CNT4 - 1) // 2, pair, carry)
        wait_e(1)
        wait_e(0)
        ptr_lo, ptr_hi, wp_lo, wp_hi = carry

        lanes = lax.iota(jnp.int32, 16)
        for ptr, wp, h, f0 in ((ptr_lo, wp_lo, 0, 0), (ptr_hi, wp_hi, 1, 3)):
            start16 = ptr & ~15
            keep = lanes < (ptr - start16)
            for f in range(3):
                vcur = st[f0 + f, pl.ds(start16, 16)]
                st[f0 + f, pl.ds(start16, 16)] = jnp.where(keep, vcur, zero16)
                for j in range(1, 8):
                    st[f0 + f, pl.ds(start16 + j * 16, 16)] = zero16
            nrem = (ptr + C - 1) // C

            def dflush(j, carry2, wp=wp, h=h, f0=f0):
                off = pl.multiple_of(base[h] + wp + j * C, 8)
                for f in range(3):
                    pltpu.sync_copy(
                        st.at[f0 + f, pl.ds(j * C, C)],
                        outs[f].at[h, pl.ds(off, C)])
                return carry2
            lax.fori_loop(0, nrem, dflush, 0)

        for h in range(2):
            st_h = _odd_steps(tot[h])
            npad = st_h * 2 * NS - tot[h]

            def pchunk(j, carry2, h=h, npad=npad):
                pc = pl.multiple_of((tot[h] + me + j * NW) * C, 8)
                for f in range(3):
                    pltpu.sync_copy(zc, outs[f].at[h, pl.ds(pc, C)])
                return carry2
            npad_mine = jnp.maximum((npad - me + NW - 1) // NW, 0)
            lax.fori_loop(0, npad_mine, pchunk, 0)

    return k(pk, counts)


def _propagate(fd, fs, fw, counts, cur):
    @functools.partial(
        pl.kernel,
        out_type=jax.ShapeDtypeStruct((N, D), jnp.float32),
        mesh=_mesh,
        scratch_types=[
            pltpu.VMEM_SHARED((HALF, D), jnp.float32),
            pltpu.VMEM((ZROWS, D), jnp.float32),
            pltpu.VMEM((2, 3, 2, C), jnp.int32),
            pltpu.VMEM((2, NW, 16), jnp.int32),
            pltpu.VMEM((2, 2, C, D), jnp.float32),
            pltpu.SemaphoreType.DMA,
            pltpu.SemaphoreType.DMA,
            pltpu.SemaphoreType.DMA,
            pltpu.SemaphoreType.DMA,
            pltpu.SemaphoreType.DMA,
            pltpu.SemaphoreType.DMA,
        ],
        compiler_params=_params,
    )
    def k(fd_hbm, fs_hbm, fw_hbm, counts_hbm, cur_hbm, out_hbm,
          accum, zbuf, ebuf, cbuf, rows,
          sem_e0, sem_e1, sem_g0, sem_g1, sem_s0, sem_s1):
        cid = lax.axis_index("c")
        sid = lax.axis_index("s")
        sem_e = (sem_e0, sem_e1)
        sem_g = (sem_g0, sem_g1)
        sem_s = (sem_s0, sem_s1)
        zero16 = jnp.zeros((16,), jnp.float32)

        pltpu.sync_copy(counts_hbm, cbuf)
        tot = jnp.zeros((16,), jnp.int32)
        for t in range(NW):
            tot = tot + cbuf[cid, t, pl.ds(0, 16)]
        m = _odd_steps(jnp.max(tot))

        def cbase(i):
            return jnp.minimum(sid * 2 + i * 2 * NS, CAPC - 2)

        def issue_ecopy(i, p):
            cb = cbase(i)
            pltpu.async_copy(fd_hbm.at[cid, pl.ds(cb, 2)], ebuf.at[p, 0], sem_e[p])
            pltpu.async_copy(fs_hbm.at[cid, pl.ds(cb, 2)], ebuf.at[p, 1], sem_e[p])
            pltpu.async_copy(fw_hbm.at[cid, pl.ds(cb, 2)], ebuf.at[p, 2], sem_e[p])

        def wait_ecopy(p):
            for j in range(3):
                pltpu.make_async_copy(fd_hbm.at[0, pl.ds(0, 2)],
                                      ebuf.at[p, j], sem_e[p]).wait()

        def issue_gather(p):
            for h in range(2):
                pltpu.async_copy(cur_hbm.at[ebuf.at[p, 1, h]], rows.at[p, h],
                                 sem_g[p])

        def wait_gather_h(p, h):
            pltpu.make_async_copy(cur_hbm.at[ebuf.at[p, 1, h]],
                                  rows.at[p, h], sem_g[p]).wait()

        def issue_scatter_h(p, h):
            pltpu.async_copy(rows.at[p, h], accum.at[ebuf.at[p, 0, h]],
                             sem_s[p], add=True)

        def wait_gather(p):
            for h in range(2):
                wait_gather_h(p, h)

        def issue_scatter(p):
            for h in range(2):
                issue_scatter_h(p, h)

        def wait_scatter(p):
            for h in range(2):
                pltpu.make_async_copy(rows.at[p, h], accum.at[ebuf.at[p, 0, h]],
                                      sem_s[p]).wait()

        def process_h(p, h):
            for v in range(C // 16):
                wg = lax.bitcast_convert_type(
                    ebuf[p, 2, h, pl.ds(v * 16, 16)], jnp.float32)
                for t in range(16):
                    j = v * 16 + t
                    ws = _lane_bcast(wg, t)
                    rows[p, h, j, pl.ds(0, 16)] = (
                        rows[p, h, j, pl.ds(0, 16)] * ws)
                    rows[p, h, j, pl.ds(16, 16)] = (
                        rows[p, h, j, pl.ds(16, 16)] * ws)

        def process(p):
            for h in range(2):
                process_h(p, h)

        def zb(i, carry):
            zbuf[i, pl.ds(0, 16)] = zero16
            zbuf[i, pl.ds(16, 16)] = zero16
            return carry
        lax.fori_loop(0, ZROWS, zb, 0)

        nblocks_mine = (NBLOCKS - sid + NS - 1) // NS

        def zero_block(i, carry):
            pltpu.sync_copy(zbuf, accum.at[pl.ds((sid + i * NS) * ZROWS, ZROWS)])
            return carry
        lax.fori_loop(0, nblocks_mine, zero_block, 0)

        issue_ecopy(0, 0)
        wait_ecopy(0)
        issue_gather(0)
        issue_ecopy(1, 1)
        plsc.subcore_barrier()

        def halfstep(i, p, scatter_wait_cond=None):
            wait_ecopy(p)
            if scatter_wait_cond is None:
                wait_scatter(p)
            else:
                @pl.when(scatter_wait_cond)
                def _():
                    wait_scatter(p)
            issue_gather(p)
            for h in range(2):
                wait_gather_h(1 - p, h)
                process_h(1 - p, h)
                issue_scatter_h(1 - p, h)
            issue_ecopy(i + 1, 1 - p)

        def pair(jj, carry):
            i1 = 2 * jj + 1
            halfstep(i1, 1, scatter_wait_cond=jj > 0)
            halfstep(i1 + 1, 0)
            return carry
        lax.fori_loop(0, (m - 1) // 2, pair, 0)

        wait_gather(0)
        process(0)
        issue_scatter(0)
        wait_ecopy(1)

        @pl.when(m > 1)
        def _():
            wait_scatter(1)

        wait_scatter(0)
        plsc.subcore_barrier()

        def write_block(i, carry):
            r0 = (sid + i * NS) * ZROWS
            pltpu.sync_copy(accum.at[pl.ds(r0, ZROWS)],
                            out_hbm.at[pl.ds(cid * HALF + r0, ZROWS)])
            return carry
        lax.fori_loop(0, nblocks_mine, write_block, 0)

    return k(fd, fs, fw, counts, cur)


def _score(pid, hid, t0, t1, t2, t3):
    @functools.partial(
        pl.kernel,
        out_type=jax.ShapeDtypeStruct((B,), jnp.float32),
        mesh=_mesh,
        scratch_types=[
            pltpu.VMEM((PP,), jnp.int32),
            pltpu.VMEM((PP,), jnp.int32),
            pltpu.VMEM((PP, D), jnp.float32),
            pltpu.VMEM((PP, D), jnp.float32),
            pltpu.VMEM((PP, D), jnp.float32),
            pltpu.VMEM((PP,), jnp.float32),
            pltpu.SemaphoreType.DMA,
        ],
        compiler_params=_params,
    )
    def k(pid_hbm, hid_hbm, t0_hbm, t1_hbm, t2_hbm, t3_hbm, out_hbm,
          pidb, hidb, pacc, hacc, tmp, sb, sem):
        cid = lax.axis_index("c")
        sid = lax.axis_index("s")
        base = (cid * NS + sid) * PP
        pltpu.sync_copy(pid_hbm.at[pl.ds(base, PP)], pidb)
        pltpu.sync_copy(hid_hbm.at[pl.ds(base, PP)], hidb)

        def off(i, carry):
            hidb[pl.ds(i * 16, 16)] = hidb[pl.ds(i * 16, 16)] + NUM_P
            return carry
        lax.fori_loop(0, PP // 16, off, 0)

        def accumulate(idxb, acc):
            pltpu.async_copy(t0_hbm.at[idxb], acc, sem).wait()
            for t_hbm in (t1_hbm, t2_hbm, t3_hbm):
                pltpu.async_copy(t_hbm.at[idxb], tmp, sem).wait()

                def addv(r, carry):
                    acc[r, pl.ds(0, 16)] = acc[r, pl.ds(0, 16)] + tmp[r, pl.ds(0, 16)]
                    acc[r, pl.ds(16, 16)] = acc[r, pl.ds(16, 16)] + tmp[r, pl.ds(16, 16)]
                    return carry
                lax.fori_loop(0, PP, addv, 0)

        accumulate(pidb, pacc)
        accumulate(hidb, hacc)

        iota = lax.iota(jnp.int32, 16)

        def dotg(g, carry):
            rowidx = g * 16 + iota
            acc = jnp.zeros((16,), jnp.float32)
            for d in range(D):
                col = jnp.full((16,), d, jnp.int32)
                pc = plsc.load_gather(pacc, [rowidx, col])
                hc = plsc.load_gather(hacc, [rowidx, col])
                acc = acc + pc * hc
            sb[pl.ds(g * 16, 16)] = acc * jnp.float32(1.0 / 16.0)
            return carry
        lax.fori_loop(0, PP // 16, dotg, 0)
        pltpu.sync_copy(sb, out_hbm.at[pl.ds(base, PP)])

    return k(pid, hid, t0, t1, t2, t3)


def kernel(person_ids, hobby_ids, edge_index, edge_weight, person_emb, hobby_emb):
    dst = edge_index[0].astype(jnp.int32)
    src = edge_index[1].astype(jnp.int32)
    w = edge_weight.astype(jnp.float32)
    pk = jnp.stack([dst.reshape(NCHUNKS, C),
                    src.reshape(NCHUNKS, C),
                    lax.bitcast_convert_type(w, jnp.int32).reshape(NCHUNKS, C)],
                   axis=1)
    counts = _count(dst.reshape(NCHUNKS, C))
    fd, fs, fw = _compact(pk, counts)
    fd = fd.reshape(2, CAPC, C)
    fs = fs.reshape(2, CAPC, C)
    fw = fw.reshape(2, CAPC, C)
    t0 = jnp.concatenate([person_emb, hobby_emb], axis=0)
    t1 = _propagate(fd, fs, fw, counts, t0)
    t2 = _propagate(fd, fs, fw, counts, t1)
    t3 = _propagate(fd, fs, fw, counts, t2)
    return _score(person_ids.astype(jnp.int32), hobby_ids.astype(jnp.int32),
                  t0, t1, t2, t3)
